# Initial kernel scaffold; baseline (speedup 1.0000x reference)
import jax
import jax.numpy as jnp
from jax.experimental import pallas as pl


def kernel(x, edge_index, edge_attr, params):
    raise NotImplementedError("write your pallas kernel here")


# speedup vs baseline: 1.3612x; 1.3612x over previous
import functools

import jax
import jax.numpy as jnp
from jax.experimental import pallas as pl
from jax.experimental.pallas import tpu as pltpu

H = 64
LAT = 128
F32 = jnp.float32


def _dot(a, b):
    return jnp.dot(a, b, preferred_element_type=F32)


def _enc_body(x_ref, w0t, b0, w1t, b1, wrt, wct, h_ref, a_ref, b_ref):
    h = jnp.maximum(_dot(x_ref[...], w0t[...]) + b0[...], 0.0)
    h = _dot(h, w1t[...]) + b1[...]
    h_ref[...] = h
    a_ref[...] = _dot(h, wrt[...])
    b_ref[...] = _dot(h, wct[...])


def _edge_body(ga_ref, gb_ref, ea_ref, w0et, b0, w1t, b1, w2t, b2, out_ref):
    e = ga_ref[...] + gb_ref[...] + _dot(ea_ref[...], w0et[...]) + b0[...]
    e = jnp.maximum(e, 0.0)
    e = jnp.maximum(_dot(e, w1t[...]) + b1[...], 0.0)
    out_ref[...] = _dot(e, w2t[...]) + b2[...]


def _node_common(h_ref, ag_ref, w0ht, w0at, b0, w1t, b1, g, be):
    h = h_ref[...]
    n = jnp.maximum(_dot(h, w0ht[...]) + _dot(ag_ref[...], w0at[...]) + b0[...], 0.0)
    n = _dot(n, w1t[...]) + b1[...]
    hn = h + n
    mu = jnp.mean(hn, axis=-1, keepdims=True)
    v = jnp.mean((hn - mu) ** 2, axis=-1, keepdims=True)
    return (hn - mu) / jnp.sqrt(v + 1e-5) * g[...] + be[...]


def _node_proj_body(h_ref, ag_ref, w0ht, w0at, b0, w1t, b1, g, be, wrt, wct,
                    h_out, a_out, b_out):
    hnew = _node_common(h_ref, ag_ref, w0ht, w0at, b0, w1t, b1, g, be)
    h_out[...] = hnew
    a_out[...] = _dot(hnew, wrt[...])
    b_out[...] = _dot(hnew, wct[...])


def _node_last_body(h_ref, ag_ref, w0ht, w0at, b0, w1t, b1, g, be, h_out):
    h_out[...] = _node_common(h_ref, ag_ref, w0ht, w0at, b0, w1t, b1, g, be)


def _sig(x):
    return jax.nn.sigmoid(x)


def _lstm_cell(gates, c):
    i, f, g, o = jnp.split(gates, 4, axis=-1)
    c = _sig(f) * c + _sig(i) * jnp.tanh(g)
    return _sig(o) * jnp.tanh(c), c


def _head(h, w0t, b0, w1t, b1):
    return _dot(jnp.maximum(_dot(h, w0t[...]) + b0[...], 0.0), w1t[...]) + b1[...]


def _lstm_head_body(x0_ref, x1_ref,
                    wih1t, whh1t, bb1, wih2t, whh2t, bb2,
                    sw0t, sb0, sw1t, sb1, dw0t, db0, dw1t, db1,
                    pw0t, pb0, pw1t, pb1,
                    s_out, d_out, p_out):
    g0 = _dot(x0_ref[...], wih1t[...]) + bb1[...]
    y0, c = _lstm_cell(g0, 0.0)
    g1 = _dot(x1_ref[...], wih1t[...]) + _dot(y0, whh1t[...]) + bb1[...]
    y1, c = _lstm_cell(g1, c)
    g0 = _dot(y0, wih2t[...]) + bb2[...]
    z0, c = _lstm_cell(g0, 0.0)
    g1 = _dot(y1, wih2t[...]) + _dot(z0, whh2t[...]) + bb2[...]
    z1, c = _lstm_cell(g1, c)
    s_out[...] = _head(z1, sw0t, sb0, sw1t, sb1)
    d = _head(z1, dw0t, db0, dw1t, db1)
    nrm = jnp.sqrt(jnp.sum(d * d, axis=-1, keepdims=True))
    d_out[...] = d / jnp.maximum(nrm, 1e-12)
    p_out[...] = _head(z1, pw0t, pb0, pw1t, pb1)


def _full_spec(shape):
    return pl.BlockSpec(shape, lambda i: tuple(0 for _ in shape))


def _row_spec(bn, cols):
    return pl.BlockSpec((bn, cols), lambda i: (i, 0))


def _tc_call(body, grid, in_specs, out_specs, out_shapes):
    return pl.pallas_call(
        body,
        grid=(grid,),
        in_specs=in_specs,
        out_specs=out_specs,
        out_shape=out_shapes,
    )


def _r2(v):
    return v.reshape(1, -1)


def kernel(x, edge_index, edge_attr, params):
    Bsz, T, N, INP = x.shape
    E = edge_index.shape[1]
    row, col = edge_index[0], edge_index[1]
    p = params

    xs = x.reshape(Bsz * T * N, INP)
    lyr0 = p['layers'][0]

    BN = 2000
    nblk = (Bsz * T * N) // BN
    enc_in = [
        _row_spec(BN, INP),
        _full_spec((INP, H)), _full_spec((1, H)),
        _full_spec((H, H)), _full_spec((1, H)),
        _full_spec((H, H)), _full_spec((H, H)),
    ]
    enc_out = [_row_spec(BN, H)] * 3
    h_ab = _tc_call(
        _enc_body, nblk, enc_in, enc_out,
        [jax.ShapeDtypeStruct((Bsz * T * N, H), F32)] * 3,
    )(xs, p['encW0'].T, _r2(p['encb0']), p['encW1'].T, _r2(p['encb1']),
      lyr0['eW0'][:, :H].T, lyr0['eW0'][:, H:2 * H].T)
    hh, aa, bb = h_ab
    hs = [hh[:N], hh[N:]]
    a_s = [aa[:N], aa[N:]]
    b_s = [bb[:N], bb[N:]]

    EDIM = edge_attr.shape[1]
    ea8 = jnp.pad(edge_attr, ((0, 0), (0, 8 - EDIM)))

    BE = 8000
    eblk = E // BE

    def edge_mlp(ga, gb, lyr):
        w0et = jnp.pad(lyr['eW0'][:, 2 * H:].T, ((0, 8 - EDIM), (0, 0)))
        specs = [
            _row_spec(BE, H), _row_spec(BE, H), _row_spec(BE, 8),
            _full_spec((8, H)), _full_spec((1, H)),
            _full_spec((H, H)), _full_spec((1, H)),
            _full_spec((H, H)), _full_spec((1, H)),
        ]
        return _tc_call(
            _edge_body, eblk, specs, _row_spec(BE, H),
            jax.ShapeDtypeStruct((E, H), F32),
        )(ga, gb, ea8, w0et, _r2(lyr['eb0']), lyr['eW1'].T, _r2(lyr['eb1']),
          lyr['eW2'].T, _r2(lyr['eb2']))

    def node_update(h, ag, lyr, nxt):
        w_common = (lyr['nW0'][:, :H].T, lyr['nW0'][:, H:].T, _r2(lyr['nb0']),
                    lyr['nW1'].T, _r2(lyr['nb1']), _r2(lyr['g']), _r2(lyr['be']))
        specs_common = [
            _row_spec(BN, H), _row_spec(BN, H),
            _full_spec((H, H)), _full_spec((H, H)), _full_spec((1, H)),
            _full_spec((H, H)), _full_spec((1, H)),
            _full_spec((1, H)), _full_spec((1, H)),
        ]
        if nxt is None:
            return _tc_call(
                _node_last_body, N // BN, specs_common, _row_spec(BN, H),
                jax.ShapeDtypeStruct((N, H), F32),
            )(h, ag, *w_common)
        specs = specs_common + [_full_spec((H, H)), _full_spec((H, H))]
        return _tc_call(
            _node_proj_body, N // BN, specs, [_row_spec(BN, H)] * 3,
            [jax.ShapeDtypeStruct((N, H), F32)] * 3,
        )(h, ag, *w_common, nxt['eW0'][:, :H].T, nxt['eW0'][:, H:2 * H].T)

    NL = len(p['layers'])
    for li, lyr in enumerate(p['layers']):
        nxt = p['layers'][li + 1] if li + 1 < NL else None
        for t in range(T):
            ga = a_s[t][row]
            gb = b_s[t][col]
            e2 = edge_mlp(ga, gb, lyr)
            ag = jnp.zeros((N, H), F32).at[col].add(e2)
            if nxt is None:
                hs[t] = node_update(hs[t], ag, lyr, None)
            else:
                hs[t], a_s[t], b_s[t] = node_update(hs[t], ag, lyr, nxt)

    lp1, lp2 = p['lstm']
    specs = [
        _row_spec(BN, H), _row_spec(BN, H),
        _full_spec((H, 4 * LAT)), _full_spec((LAT, 4 * LAT)), _full_spec((1, 4 * LAT)),
        _full_spec((LAT, 4 * LAT)), _full_spec((LAT, 4 * LAT)), _full_spec((1, 4 * LAT)),
        _full_spec((LAT, H // 2)), _full_spec((1, H // 2)), _full_spec((H // 2, 1)), _full_spec((1, 1)),
        _full_spec((LAT, H // 2)), _full_spec((1, H // 2)), _full_spec((H // 2, 2)), _full_spec((1, 2)),
        _full_spec((LAT, H // 2)), _full_spec((1, H // 2)), _full_spec((H // 2, 1)), _full_spec((1, 1)),
    ]
    out_specs = [_row_spec(BN, 1), _row_spec(BN, 2), _row_spec(BN, 1)]
    out_shapes = [jax.ShapeDtypeStruct((N, 1), F32),
                  jax.ShapeDtypeStruct((N, 2), F32),
                  jax.ShapeDtypeStruct((N, 1), F32)]
    s, d, pp = _tc_call(_lstm_head_body, N // BN, specs, out_specs, out_shapes)(
        hs[0], hs[1],
        lp1['Wih'].T, lp1['Whh'].T, _r2(lp1['bih'] + lp1['bhh']),
        lp2['Wih'].T, lp2['Whh'].T, _r2(lp2['bih'] + lp2['bhh']),
        p['sW0'].T, _r2(p['sb0']), p['sW1'].T, _r2(p['sb1']),
        p['dW0'].T, _r2(p['db0']), p['dW1'].T, _r2(p['db1']),
        p['pW0'].T, _r2(p['pb0']), p['pW1'].T, _r2(p['pb1']),
    )
    return jnp.concatenate([s, d, pp], axis=-1).reshape(Bsz, N, 4)


# speedup vs baseline: 3.5705x; 2.6230x over previous
import functools

import jax
import jax.numpy as jnp
from jax import lax
from jax.experimental import pallas as pl
from jax.experimental.pallas import tpu as pltpu
from jax.experimental.pallas import tpu_sc as plsc

H = 64
H2 = 128
LAT = 128
F32 = jnp.float32
BF16 = jnp.bfloat16

NC, NS = 2, 16
NW = NC * NS
CH = 128
DEPTH = 3

N_NODES = 10000
E_EDGES = 160000
EP = 163840
NCH = EP // (NW * CH)
NP = 10240
NPT = NP // NS


def _dot(a, b):
    return jnp.dot(a, b, preferred_element_type=F32)


def _gather_body(a_hbm, b_hbm, row_hbm, col_hbm, ga_hbm, gb_hbm,
                 rowv, colv, bufa, bufb, sema, semb):
    wid = lax.axis_index("s") * NC + lax.axis_index("c")
    cbase = wid * NCH
    pltpu.sync_copy(row_hbm.at[pl.ds(cbase, NCH)], rowv)
    pltpu.sync_copy(col_hbm.at[pl.ds(cbase, NCH)], colv)

    for d in range(DEPTH - 1):
        pltpu.async_copy(a_hbm.at[rowv.at[d]], bufa.at[d], sema.at[d])
        pltpu.async_copy(b_hbm.at[colv.at[d]], bufb.at[d], semb.at[d])

    @pl.loop(0, NCH)
    def _(j):
        slot = lax.rem(j, DEPTH)
        pltpu.make_async_copy(a_hbm.at[rowv.at[j]], bufa.at[slot], sema.at[slot]).wait()
        pltpu.make_async_copy(b_hbm.at[colv.at[j]], bufb.at[slot], semb.at[slot]).wait()
        base = (cbase + j) * CH
        pltpu.sync_copy(bufa.at[slot], ga_hbm.at[pl.ds(base, CH)])
        pltpu.sync_copy(bufb.at[slot], gb_hbm.at[pl.ds(base, CH)])
        jn = j + DEPTH - 1

        @pl.when(jn < NCH)
        def _():
            ns = lax.rem(jn, DEPTH)
            pltpu.async_copy(a_hbm.at[rowv.at[jn]], bufa.at[ns], sema.at[ns])
            pltpu.async_copy(b_hbm.at[colv.at[jn]], bufb.at[ns], semb.at[ns])


@functools.lru_cache(maxsize=None)
def _gather_call():
    return pl.kernel(
        _gather_body,
    out_type=[jax.ShapeDtypeStruct((EP, H2), F32)] * 2,
    mesh=plsc.VectorSubcoreMesh(core_axis_name="c", subcore_axis_name="s"),
    scratch_types=[
        pltpu.VMEM((NCH, CH), jnp.int32),
        pltpu.VMEM((NCH, CH), jnp.int32),
        pltpu.VMEM((DEPTH, CH, H2), F32),
        pltpu.VMEM((DEPTH, CH, H2), F32),
            pltpu.SemaphoreType.DMA((DEPTH,)),
            pltpu.SemaphoreType.DMA((DEPTH,)),
        ],
    )


DEPTH_S = 2


def _scatter_body(e2_hbm, col_hbm, zero_hbm, out_hbm, colv, bufe, aggr, sem):
    c = lax.axis_index("c")
    s = lax.axis_index("s")
    wid = s * NC + c
    cbase = wid * NCH
    pltpu.sync_copy(col_hbm.at[pl.ds(cbase, NCH)], colv)
    pltpu.sync_copy(zero_hbm.at[pl.ds(s * NPT, NPT)], aggr.at[pl.ds(s * NPT, NPT)])
    plsc.subcore_barrier()

    for d in range(DEPTH_S - 1):
        pltpu.async_copy(e2_hbm.at[pl.ds((cbase + d) * CH, CH)], bufe.at[d], sem.at[d])

    @pl.loop(0, NCH)
    def _(j):
        slot = lax.rem(j, DEPTH_S)
        pltpu.make_async_copy(
            e2_hbm.at[pl.ds((cbase + j) * CH, CH)], bufe.at[slot], sem.at[slot]
        ).wait()
        pltpu.sync_copy(bufe.at[slot], aggr.at[colv.at[j]], add=True)
        jn = j + DEPTH_S - 1

        @pl.when(jn < NCH)
        def _():
            ns = lax.rem(jn, DEPTH_S)
            pltpu.async_copy(e2_hbm.at[pl.ds((cbase + jn) * CH, CH)], bufe.at[ns], sem.at[ns])

    plsc.subcore_barrier()
    pltpu.sync_copy(aggr.at[pl.ds(s * NPT, NPT)], out_hbm.at[c, pl.ds(s * NPT, NPT)])


@functools.lru_cache(maxsize=None)
def _scatter_call():
    return pl.kernel(
        _scatter_body,
        out_type=jax.ShapeDtypeStruct((NC, NP, H2), F32),
        mesh=plsc.VectorSubcoreMesh(core_axis_name="c", subcore_axis_name="s"),
        scratch_types=[
            pltpu.VMEM((NCH, CH), jnp.int32),
            pltpu.VMEM((DEPTH_S, CH, H2), F32),
            pltpu.VMEM_SHARED((NP, H2), F32),
            pltpu.SemaphoreType.DMA((DEPTH_S,)),
        ],
    )


def _sc_gather(a2, b2, row2d, col2d):
    return _gather_call()(a2, b2, row2d, col2d)


def _sc_scatter(e2, col2d, zeros_nh):
    return _scatter_call()(e2, col2d, zeros_nh)


def _enc_body(x0_ref, x1_ref, w0t, b0, w1t, b1, wrt2, wct2, h_ref, a_ref, b_ref):
    def enc(xr):
        h = jnp.maximum(_dot(xr[...], w0t[...]) + b0[...], 0.0)
        return _dot(h, w1t[...]) + b1[...]

    h2 = jnp.concatenate([enc(x0_ref), enc(x1_ref)], axis=-1)
    h_ref[...] = h2
    a_ref[...] = _dot(h2, wrt2[...])
    b_ref[...] = _dot(h2, wct2[...])


def _edge_body(ga_ref, gb_ref, ea_ref, w0et2, b02, w1t2, b12, w2t2, b22,
               out_ref, *, be):
    e = ga_ref[...] + gb_ref[...] + _dot(ea_ref[...], w0et2[...]) + b02[...]
    e = jnp.maximum(e, 0.0)
    e = jnp.maximum(_dot(e, w1t2[...]) + b12[...], 0.0)
    e = _dot(e, w2t2[...]) + b22[...]
    gidx = lax.broadcasted_iota(jnp.int32, (be, 1), 0) + pl.program_id(0) * be
    out_ref[...] = jnp.where(gidx < E_EDGES, e, 0.0)


def _ln_half(hn, g, be):
    mu = jnp.mean(hn, axis=-1, keepdims=True)
    v = jnp.mean((hn - mu) ** 2, axis=-1, keepdims=True)
    return (hn - mu) / jnp.sqrt(v + 1e-5) * g + be


def _node_common(h_ref, ag0_ref, ag1_ref, w0ht2, w0at2, b02, w1t2, b12, g, be):
    h2 = h_ref[...]
    ag = ag0_ref[0] + ag1_ref[0]
    n = jnp.maximum(_dot(h2, w0ht2[...]) + _dot(ag, w0at2[...]) + b02[...], 0.0)
    n = _dot(n, w1t2[...]) + b12[...]
    hn = h2 + n
    return jnp.concatenate(
        [_ln_half(hn[:, :H], g[...], be[...]),
         _ln_half(hn[:, H:], g[...], be[...])], axis=-1)


def _node_proj_body(h_ref, ag0_ref, ag1_ref, w0ht2, w0at2, b02, w1t2, b12, g, be,
                    wrt2, wct2, h_out, a_out, b_out):
    hnew = _node_common(h_ref, ag0_ref, ag1_ref, w0ht2, w0at2, b02, w1t2, b12, g, be)
    h_out[...] = hnew
    a_out[...] = _dot(hnew, wrt2[...])
    b_out[...] = _dot(hnew, wct2[...])


def _node_last_body(h_ref, ag0_ref, ag1_ref, w0ht2, w0at2, b02, w1t2, b12, g, be,
                    h_out):
    h_out[...] = _node_common(h_ref, ag0_ref, ag1_ref, w0ht2, w0at2, b02, w1t2,
                              b12, g, be)


def _sig(x):
    return jax.nn.sigmoid(x)


def _lstm_cell(gates, c):
    i, f, g, o = jnp.split(gates, 4, axis=-1)
    c = _sig(f) * c + _sig(i) * jnp.tanh(g)
    return _sig(o) * jnp.tanh(c), c


def _head(h, w0t, b0, w1t, b1):
    return _dot(jnp.maximum(_dot(h, w0t[...]) + b0[...], 0.0), w1t[...]) + b1[...]


def _lstm_head_body(h2_ref,
                    wih1t, whh1t, bb1, wih2t, whh2t, bb2,
                    sw0t, sb0, sw1t, sb1, dw0t, db0, dw1t, db1,
                    pw0t, pb0, pw1t, pb1,
                    s_out, d_out, p_out):
    x0 = h2_ref[:, :H]
    x1 = h2_ref[:, H:]
    g0 = _dot(x0, wih1t[...]) + bb1[...]
    y0, c = _lstm_cell(g0, 0.0)
    g1 = _dot(x1, wih1t[...]) + _dot(y0, whh1t[...]) + bb1[...]
    y1, c = _lstm_cell(g1, c)
    g0 = _dot(y0, wih2t[...]) + bb2[...]
    z0, c = _lstm_cell(g0, 0.0)
    g1 = _dot(y1, wih2t[...]) + _dot(z0, whh2t[...]) + bb2[...]
    z1, c = _lstm_cell(g1, c)
    s_out[...] = _head(z1, sw0t, sb0, sw1t, sb1)
    d = _head(z1, dw0t, db0, dw1t, db1)
    nrm = jnp.sqrt(jnp.sum(d * d, axis=-1, keepdims=True))
    d_out[...] = d / jnp.maximum(nrm, 1e-12)
    p_out[...] = _head(z1, pw0t, pb0, pw1t, pb1)


def _full_spec(shape):
    return pl.BlockSpec(shape, lambda i: tuple(0 for _ in shape))


def _row_spec(bn, cols):
    return pl.BlockSpec((bn, cols), lambda i: (i, 0))


def _tc_call(body, grid, in_specs, out_specs, out_shapes):
    return pl.pallas_call(
        body,
        grid=(grid,),
        in_specs=in_specs,
        out_specs=out_specs,
        out_shape=out_shapes,
    )


def _r2(v):
    return v.reshape(1, -1)


def _bd(wt):
    k, m = wt.shape
    z = jnp.zeros((k, m), wt.dtype)
    return jnp.concatenate(
        [jnp.concatenate([wt, z], axis=1), jnp.concatenate([z, wt], axis=1)], axis=0)


def _p2(v):
    return jnp.concatenate([v, v]).reshape(1, -1)


def kernel(x, edge_index, edge_attr, params):
    Bsz, T, N, INP = x.shape
    E = edge_index.shape[1]
    p = params
    pad = EP - E
    row2d = jnp.pad(edge_index[0], (0, pad)).reshape(EP // CH, CH)
    col2d = jnp.pad(edge_index[1], (0, pad)).reshape(EP // CH, CH)
    zeros_nh = jnp.zeros((NP, H2), F32)

    xs = x.reshape(Bsz * T * N, INP)
    lyr0 = p['layers'][0]

    BN = 2000
    nblk = N // BN
    enc_in = [
        pl.BlockSpec((BN, INP), lambda i: (i, 0)),
        pl.BlockSpec((BN, INP), lambda i: (i + nblk, 0)),
        _full_spec((INP, H)), _full_spec((1, H)),
        _full_spec((H, H)), _full_spec((1, H)),
        _full_spec((H2, H2)), _full_spec((H2, H2)),
    ]
    enc_out = [_row_spec(BN, H2)] * 3
    h2, a2, b2 = _tc_call(
        _enc_body, nblk, enc_in, enc_out,
        [jax.ShapeDtypeStruct((N, H2), F32)] * 3,
    )(xs, xs, p['encW0'].T, _r2(p['encb0']), p['encW1'].T, _r2(p['encb1']),
      _bd(lyr0['eW0'][:, :H].T), _bd(lyr0['eW0'][:, H:2 * H].T))

    EDIM = edge_attr.shape[1]
    ea8 = jnp.pad(edge_attr, ((0, pad), (0, 8 - EDIM)))

    BE = 4096
    eblk = EP // BE

    def edge_mlp(ga, gb, lyr):
        w0et = jnp.pad(lyr['eW0'][:, 2 * H:].T, ((0, 8 - EDIM), (0, 0)))
        w0et2 = jnp.concatenate([w0et, w0et], axis=1)
        specs = [
            _row_spec(BE, H2), _row_spec(BE, H2), _row_spec(BE, 8),
            _full_spec((8, H2)), _full_spec((1, H2)),
            _full_spec((H2, H2)), _full_spec((1, H2)),
            _full_spec((H2, H2)), _full_spec((1, H2)),
        ]
        return _tc_call(
            functools.partial(_edge_body, be=BE), eblk, specs, _row_spec(BE, H2),
            jax.ShapeDtypeStruct((EP, H2), F32),
        )(ga, gb, ea8, w0et2, _p2(lyr['eb0']), _bd(lyr['eW1'].T), _p2(lyr['eb1']),
          _bd(lyr['eW2'].T), _p2(lyr['eb2']))

    def node_update(h2c, parts, lyr, nxt):
        w_common = (_bd(lyr['nW0'][:, :H].T), _bd(lyr['nW0'][:, H:].T), _p2(lyr['nb0']),
                    _bd(lyr['nW1'].T), _p2(lyr['nb1']), _r2(lyr['g']), _r2(lyr['be']))
        ag0_spec = pl.BlockSpec((1, BN, H2), lambda i: (0, i, 0))
        ag1_spec = pl.BlockSpec((1, BN, H2), lambda i: (1, i, 0))
        specs_common = [
            _row_spec(BN, H2), ag0_spec, ag1_spec,
            _full_spec((H2, H2)), _full_spec((H2, H2)), _full_spec((1, H2)),
            _full_spec((H2, H2)), _full_spec((1, H2)),
            _full_spec((1, H)), _full_spec((1, H)),
        ]
        if nxt is None:
            return _tc_call(
                _node_last_body, N // BN, specs_common, _row_spec(BN, H2),
                jax.ShapeDtypeStruct((N, H2), F32),
            )(h2c, parts, parts, *w_common)
        specs = specs_common + [_full_spec((H2, H2)), _full_spec((H2, H2))]
        return _tc_call(
            _node_proj_body, N // BN, specs, [_row_spec(BN, H2)] * 3,
            [jax.ShapeDtypeStruct((N, H2), F32)] * 3,
        )(h2c, parts, parts, *w_common,
          _bd(nxt['eW0'][:, :H].T), _bd(nxt['eW0'][:, H:2 * H].T))

    NL = len(p['layers'])
    for li, lyr in enumerate(p['layers']):
        nxt = p['layers'][li + 1] if li + 1 < NL else None
        ga, gb = _sc_gather(a2, b2, row2d, col2d)
        e2 = edge_mlp(ga, gb, lyr)
        parts = _sc_scatter(e2, col2d, zeros_nh)
        if nxt is None:
            h2 = node_update(h2, parts, lyr, None)
        else:
            h2, a2, b2 = node_update(h2, parts, lyr, nxt)

    lp1, lp2 = p['lstm']
    specs = [
        _row_spec(BN, H2),
        _full_spec((H, 4 * LAT)), _full_spec((LAT, 4 * LAT)), _full_spec((1, 4 * LAT)),
        _full_spec((LAT, 4 * LAT)), _full_spec((LAT, 4 * LAT)), _full_spec((1, 4 * LAT)),
        _full_spec((LAT, H // 2)), _full_spec((1, H // 2)), _full_spec((H // 2, 1)), _full_spec((1, 1)),
        _full_spec((LAT, H // 2)), _full_spec((1, H // 2)), _full_spec((H // 2, 2)), _full_spec((1, 2)),
        _full_spec((LAT, H // 2)), _full_spec((1, H // 2)), _full_spec((H // 2, 1)), _full_spec((1, 1)),
    ]
    out_specs = [_row_spec(BN, 1), _row_spec(BN, 2), _row_spec(BN, 1)]
    out_shapes = [jax.ShapeDtypeStruct((N, 1), F32),
                  jax.ShapeDtypeStruct((N, 2), F32),
                  jax.ShapeDtypeStruct((N, 1), F32)]
    s, d, pp = _tc_call(_lstm_head_body, N // BN, specs, out_specs, out_shapes)(
        h2,
        lp1['Wih'].T, lp1['Whh'].T, _r2(lp1['bih'] + lp1['bhh']),
        lp2['Wih'].T, lp2['Whh'].T, _r2(lp2['bih'] + lp2['bhh']),
        p['sW0'].T, _r2(p['sb0']), p['sW1'].T, _r2(p['sb1']),
        p['dW0'].T, _r2(p['db0']), p['dW1'].T, _r2(p['db1']),
        p['pW0'].T, _r2(p['pb0']), p['pW1'].T, _r2(p['pb1']),
    )
    return jnp.concatenate([s, d, pp], axis=-1).reshape(Bsz, N, 4)


# speedup vs baseline: 3.5712x; 1.0002x over previous
import functools

import jax
import jax.numpy as jnp
from jax import lax
from jax.experimental import pallas as pl
from jax.experimental.pallas import tpu as pltpu
from jax.experimental.pallas import tpu_sc as plsc

H = 64
H2 = 128
LAT = 128
F32 = jnp.float32
BF16 = jnp.bfloat16

NC, NS = 2, 16
NW = NC * NS
CH = 128
DEPTH = 3

N_NODES = 10000
E_EDGES = 160000
EP = 163840
NCH = EP // (NW * CH)
NP = 10240
NPT = NP // NS


def _dot(a, b):
    return jnp.dot(a, b, preferred_element_type=F32)


def _gather_body(a_hbm, b_hbm, row_hbm, col_hbm, ga_hbm, gb_hbm,
                 rowv, colv, bufa, bufb, sema, semb, semoa, semob):
    wid = lax.axis_index("s") * NC + lax.axis_index("c")
    cbase = wid * NCH
    pltpu.sync_copy(row_hbm.at[pl.ds(cbase, NCH)], rowv)
    pltpu.sync_copy(col_hbm.at[pl.ds(cbase, NCH)], colv)

    for d in range(DEPTH - 1):
        pltpu.async_copy(a_hbm.at[rowv.at[d]], bufa.at[d], sema.at[d])
        pltpu.async_copy(b_hbm.at[colv.at[d]], bufb.at[d], semb.at[d])

    @pl.loop(0, NCH)
    def _(j):
        slot = lax.rem(j, DEPTH)
        pltpu.make_async_copy(a_hbm.at[rowv.at[j]], bufa.at[slot], sema.at[slot]).wait()
        pltpu.make_async_copy(b_hbm.at[colv.at[j]], bufb.at[slot], semb.at[slot]).wait()
        base = (cbase + j) * CH
        pltpu.async_copy(bufa.at[slot], ga_hbm.at[pl.ds(base, CH)], semoa.at[slot])
        pltpu.async_copy(bufb.at[slot], gb_hbm.at[pl.ds(base, CH)], semob.at[slot])
        jn = j + DEPTH - 1

        @pl.when(jn < NCH)
        def _():
            ns = lax.rem(jn, DEPTH)
            pbase = (cbase + j - 1) * CH

            @pl.when(j > 0)
            def _():
                pltpu.make_async_copy(
                    bufa.at[ns], ga_hbm.at[pl.ds(pbase, CH)], semoa.at[ns]).wait()
                pltpu.make_async_copy(
                    bufb.at[ns], gb_hbm.at[pl.ds(pbase, CH)], semob.at[ns]).wait()

            pltpu.async_copy(a_hbm.at[rowv.at[jn]], bufa.at[ns], sema.at[ns])
            pltpu.async_copy(b_hbm.at[colv.at[jn]], bufb.at[ns], semb.at[ns])

    for k in range(NCH - DEPTH, NCH):
        slot = k % DEPTH
        base = (cbase + k) * CH
        pltpu.make_async_copy(bufa.at[slot], ga_hbm.at[pl.ds(base, CH)],
                              semoa.at[slot]).wait()
        pltpu.make_async_copy(bufb.at[slot], gb_hbm.at[pl.ds(base, CH)],
                              semob.at[slot]).wait()


@functools.lru_cache(maxsize=None)
def _gather_call():
    return pl.kernel(
        _gather_body,
    out_type=[jax.ShapeDtypeStruct((EP, H2), F32)] * 2,
    mesh=plsc.VectorSubcoreMesh(core_axis_name="c", subcore_axis_name="s"),
    scratch_types=[
        pltpu.VMEM((NCH, CH), jnp.int32),
        pltpu.VMEM((NCH, CH), jnp.int32),
        pltpu.VMEM((DEPTH, CH, H2), F32),
        pltpu.VMEM((DEPTH, CH, H2), F32),
            pltpu.SemaphoreType.DMA((DEPTH,)),
            pltpu.SemaphoreType.DMA((DEPTH,)),
            pltpu.SemaphoreType.DMA((DEPTH,)),
            pltpu.SemaphoreType.DMA((DEPTH,)),
        ],
    )


DEPTH_S = 2


def _scatter_body(e2_hbm, col_hbm, zero_hbm, out_hbm, colv, bufe, aggr, sem):
    c = lax.axis_index("c")
    s = lax.axis_index("s")
    wid = s * NC + c
    cbase = wid * NCH
    pltpu.sync_copy(col_hbm.at[pl.ds(cbase, NCH)], colv)
    pltpu.sync_copy(zero_hbm.at[pl.ds(s * NPT, NPT)], aggr.at[pl.ds(s * NPT, NPT)])
    plsc.subcore_barrier()

    for d in range(DEPTH_S - 1):
        pltpu.async_copy(e2_hbm.at[pl.ds((cbase + d) * CH, CH)], bufe.at[d], sem.at[d])

    @pl.loop(0, NCH)
    def _(j):
        slot = lax.rem(j, DEPTH_S)
        pltpu.make_async_copy(
            e2_hbm.at[pl.ds((cbase + j) * CH, CH)], bufe.at[slot], sem.at[slot]
        ).wait()
        pltpu.sync_copy(bufe.at[slot], aggr.at[colv.at[j]], add=True)
        jn = j + DEPTH_S - 1

        @pl.when(jn < NCH)
        def _():
            ns = lax.rem(jn, DEPTH_S)
            pltpu.async_copy(e2_hbm.at[pl.ds((cbase + jn) * CH, CH)], bufe.at[ns], sem.at[ns])

    plsc.subcore_barrier()
    pltpu.sync_copy(aggr.at[pl.ds(s * NPT, NPT)], out_hbm.at[c, pl.ds(s * NPT, NPT)])


@functools.lru_cache(maxsize=None)
def _scatter_call():
    return pl.kernel(
        _scatter_body,
        out_type=jax.ShapeDtypeStruct((NC, NP, H2), F32),
        mesh=plsc.VectorSubcoreMesh(core_axis_name="c", subcore_axis_name="s"),
        scratch_types=[
            pltpu.VMEM((NCH, CH), jnp.int32),
            pltpu.VMEM((DEPTH_S, CH, H2), F32),
            pltpu.VMEM_SHARED((NP, H2), F32),
            pltpu.SemaphoreType.DMA((DEPTH_S,)),
        ],
    )


def _sc_gather(a2, b2, row2d, col2d):
    return _gather_call()(a2, b2, row2d, col2d)


def _sc_scatter(e2, col2d, zeros_nh):
    return _scatter_call()(e2, col2d, zeros_nh)


def _enc_body(x0_ref, x1_ref, w0t, b0, w1t, b1, wrt2, wct2, h_ref, a_ref, b_ref):
    def enc(xr):
        h = jnp.maximum(_dot(xr[...], w0t[...]) + b0[...], 0.0)
        return _dot(h, w1t[...]) + b1[...]

    h2 = jnp.concatenate([enc(x0_ref), enc(x1_ref)], axis=-1)
    h_ref[...] = h2
    a_ref[...] = _dot(h2, wrt2[...])
    b_ref[...] = _dot(h2, wct2[...])


def _edge_body(ga_ref, gb_ref, ea_ref, w0et2, b02, w1t2, b12, w2t2, b22,
               out_ref, *, be):
    e = ga_ref[...] + gb_ref[...] + _dot(ea_ref[...], w0et2[...]) + b02[...]
    e = jnp.maximum(e, 0.0)
    e = jnp.maximum(_dot(e, w1t2[...]) + b12[...], 0.0)
    e = _dot(e, w2t2[...]) + b22[...]
    gidx = lax.broadcasted_iota(jnp.int32, (be, 1), 0) + pl.program_id(0) * be
    out_ref[...] = jnp.where(gidx < E_EDGES, e, 0.0)


def _ln_half(hn, g, be):
    mu = jnp.mean(hn, axis=-1, keepdims=True)
    v = jnp.mean((hn - mu) ** 2, axis=-1, keepdims=True)
    return (hn - mu) / jnp.sqrt(v + 1e-5) * g + be


def _node_common(h_ref, ag0_ref, ag1_ref, w0ht2, w0at2, b02, w1t2, b12, g, be):
    h2 = h_ref[...]
    ag = ag0_ref[0] + ag1_ref[0]
    n = jnp.maximum(_dot(h2, w0ht2[...]) + _dot(ag, w0at2[...]) + b02[...], 0.0)
    n = _dot(n, w1t2[...]) + b12[...]
    hn = h2 + n
    return jnp.concatenate(
        [_ln_half(hn[:, :H], g[...], be[...]),
         _ln_half(hn[:, H:], g[...], be[...])], axis=-1)


def _node_proj_body(h_ref, ag0_ref, ag1_ref, w0ht2, w0at2, b02, w1t2, b12, g, be,
                    wrt2, wct2, h_out, a_out, b_out):
    hnew = _node_common(h_ref, ag0_ref, ag1_ref, w0ht2, w0at2, b02, w1t2, b12, g, be)
    h_out[...] = hnew
    a_out[...] = _dot(hnew, wrt2[...])
    b_out[...] = _dot(hnew, wct2[...])


def _node_last_body(h_ref, ag0_ref, ag1_ref, w0ht2, w0at2, b02, w1t2, b12, g, be,
                    h_out):
    h_out[...] = _node_common(h_ref, ag0_ref, ag1_ref, w0ht2, w0at2, b02, w1t2,
                              b12, g, be)


def _sig(x):
    return jax.nn.sigmoid(x)


def _lstm_cell(gates, c):
    i, f, g, o = jnp.split(gates, 4, axis=-1)
    c = _sig(f) * c + _sig(i) * jnp.tanh(g)
    return _sig(o) * jnp.tanh(c), c


def _head(h, w0t, b0, w1t, b1):
    return _dot(jnp.maximum(_dot(h, w0t[...]) + b0[...], 0.0), w1t[...]) + b1[...]


def _lstm_head_body(h2_ref,
                    wih1t, whh1t, bb1, wih2t, whh2t, bb2,
                    sw0t, sb0, sw1t, sb1, dw0t, db0, dw1t, db1,
                    pw0t, pb0, pw1t, pb1,
                    s_out, d_out, p_out):
    x0 = h2_ref[:, :H]
    x1 = h2_ref[:, H:]
    g0 = _dot(x0, wih1t[...]) + bb1[...]
    y0, c = _lstm_cell(g0, 0.0)
    g1 = _dot(x1, wih1t[...]) + _dot(y0, whh1t[...]) + bb1[...]
    y1, c = _lstm_cell(g1, c)
    g0 = _dot(y0, wih2t[...]) + bb2[...]
    z0, c = _lstm_cell(g0, 0.0)
    g1 = _dot(y1, wih2t[...]) + _dot(z0, whh2t[...]) + bb2[...]
    z1, c = _lstm_cell(g1, c)
    s_out[...] = _head(z1, sw0t, sb0, sw1t, sb1)
    d = _head(z1, dw0t, db0, dw1t, db1)
    nrm = jnp.sqrt(jnp.sum(d * d, axis=-1, keepdims=True))
    d_out[...] = d / jnp.maximum(nrm, 1e-12)
    p_out[...] = _head(z1, pw0t, pb0, pw1t, pb1)


def _full_spec(shape):
    return pl.BlockSpec(shape, lambda i: tuple(0 for _ in shape))


def _row_spec(bn, cols):
    return pl.BlockSpec((bn, cols), lambda i: (i, 0))


def _tc_call(body, grid, in_specs, out_specs, out_shapes):
    return pl.pallas_call(
        body,
        grid=(grid,),
        in_specs=in_specs,
        out_specs=out_specs,
        out_shape=out_shapes,
    )


def _r2(v):
    return v.reshape(1, -1)


def _bd(wt):
    k, m = wt.shape
    z = jnp.zeros((k, m), wt.dtype)
    return jnp.concatenate(
        [jnp.concatenate([wt, z], axis=1), jnp.concatenate([z, wt], axis=1)], axis=0)


def _p2(v):
    return jnp.concatenate([v, v]).reshape(1, -1)


def kernel(x, edge_index, edge_attr, params):
    Bsz, T, N, INP = x.shape
    E = edge_index.shape[1]
    p = params
    pad = EP - E
    row2d = jnp.pad(edge_index[0], (0, pad)).reshape(EP // CH, CH)
    col2d = jnp.pad(edge_index[1], (0, pad)).reshape(EP // CH, CH)
    zeros_nh = jnp.zeros((NP, H2), F32)

    xs = x.reshape(Bsz * T * N, INP)
    lyr0 = p['layers'][0]

    BN = 2000
    nblk = N // BN
    enc_in = [
        pl.BlockSpec((BN, INP), lambda i: (i, 0)),
        pl.BlockSpec((BN, INP), lambda i: (i + nblk, 0)),
        _full_spec((INP, H)), _full_spec((1, H)),
        _full_spec((H, H)), _full_spec((1, H)),
        _full_spec((H2, H2)), _full_spec((H2, H2)),
    ]
    enc_out = [_row_spec(BN, H2)] * 3
    h2, a2, b2 = _tc_call(
        _enc_body, nblk, enc_in, enc_out,
        [jax.ShapeDtypeStruct((N, H2), F32)] * 3,
    )(xs, xs, p['encW0'].T, _r2(p['encb0']), p['encW1'].T, _r2(p['encb1']),
      _bd(lyr0['eW0'][:, :H].T), _bd(lyr0['eW0'][:, H:2 * H].T))

    EDIM = edge_attr.shape[1]
    ea8 = jnp.pad(edge_attr, ((0, pad), (0, 8 - EDIM)))

    BE = 4096
    eblk = EP // BE

    def edge_mlp(ga, gb, lyr):
        w0et = jnp.pad(lyr['eW0'][:, 2 * H:].T, ((0, 8 - EDIM), (0, 0)))
        w0et2 = jnp.concatenate([w0et, w0et], axis=1)
        specs = [
            _row_spec(BE, H2), _row_spec(BE, H2), _row_spec(BE, 8),
            _full_spec((8, H2)), _full_spec((1, H2)),
            _full_spec((H2, H2)), _full_spec((1, H2)),
            _full_spec((H2, H2)), _full_spec((1, H2)),
        ]
        return _tc_call(
            functools.partial(_edge_body, be=BE), eblk, specs, _row_spec(BE, H2),
            jax.ShapeDtypeStruct((EP, H2), F32),
        )(ga, gb, ea8, w0et2, _p2(lyr['eb0']), _bd(lyr['eW1'].T), _p2(lyr['eb1']),
          _bd(lyr['eW2'].T), _p2(lyr['eb2']))

    def node_update(h2c, parts, lyr, nxt):
        w_common = (_bd(lyr['nW0'][:, :H].T), _bd(lyr['nW0'][:, H:].T), _p2(lyr['nb0']),
                    _bd(lyr['nW1'].T), _p2(lyr['nb1']), _r2(lyr['g']), _r2(lyr['be']))
        ag0_spec = pl.BlockSpec((1, BN, H2), lambda i: (0, i, 0))
        ag1_spec = pl.BlockSpec((1, BN, H2), lambda i: (1, i, 0))
        specs_common = [
            _row_spec(BN, H2), ag0_spec, ag1_spec,
            _full_spec((H2, H2)), _full_spec((H2, H2)), _full_spec((1, H2)),
            _full_spec((H2, H2)), _full_spec((1, H2)),
            _full_spec((1, H)), _full_spec((1, H)),
        ]
        if nxt is None:
            return _tc_call(
                _node_last_body, N // BN, specs_common, _row_spec(BN, H2),
                jax.ShapeDtypeStruct((N, H2), F32),
            )(h2c, parts, parts, *w_common)
        specs = specs_common + [_full_spec((H2, H2)), _full_spec((H2, H2))]
        return _tc_call(
            _node_proj_body, N // BN, specs, [_row_spec(BN, H2)] * 3,
            [jax.ShapeDtypeStruct((N, H2), F32)] * 3,
        )(h2c, parts, parts, *w_common,
          _bd(nxt['eW0'][:, :H].T), _bd(nxt['eW0'][:, H:2 * H].T))

    NL = len(p['layers'])
    for li, lyr in enumerate(p['layers']):
        nxt = p['layers'][li + 1] if li + 1 < NL else None
        ga, gb = _sc_gather(a2, b2, row2d, col2d)
        e2 = edge_mlp(ga, gb, lyr)
        parts = _sc_scatter(e2, col2d, zeros_nh)
        if nxt is None:
            h2 = node_update(h2, parts, lyr, None)
        else:
            h2, a2, b2 = node_update(h2, parts, lyr, nxt)

    lp1, lp2 = p['lstm']
    specs = [
        _row_spec(BN, H2),
        _full_spec((H, 4 * LAT)), _full_spec((LAT, 4 * LAT)), _full_spec((1, 4 * LAT)),
        _full_spec((LAT, 4 * LAT)), _full_spec((LAT, 4 * LAT)), _full_spec((1, 4 * LAT)),
        _full_spec((LAT, H // 2)), _full_spec((1, H // 2)), _full_spec((H // 2, 1)), _full_spec((1, 1)),
        _full_spec((LAT, H // 2)), _full_spec((1, H // 2)), _full_spec((H // 2, 2)), _full_spec((1, 2)),
        _full_spec((LAT, H // 2)), _full_spec((1, H // 2)), _full_spec((H // 2, 1)), _full_spec((1, 1)),
    ]
    out_specs = [_row_spec(BN, 1), _row_spec(BN, 2), _row_spec(BN, 1)]
    out_shapes = [jax.ShapeDtypeStruct((N, 1), F32),
                  jax.ShapeDtypeStruct((N, 2), F32),
                  jax.ShapeDtypeStruct((N, 1), F32)]
    s, d, pp = _tc_call(_lstm_head_body, N // BN, specs, out_specs, out_shapes)(
        h2,
        lp1['Wih'].T, lp1['Whh'].T, _r2(lp1['bih'] + lp1['bhh']),
        lp2['Wih'].T, lp2['Whh'].T, _r2(lp2['bih'] + lp2['bhh']),
        p['sW0'].T, _r2(p['sb0']), p['sW1'].T, _r2(p['sb1']),
        p['dW0'].T, _r2(p['db0']), p['dW1'].T, _r2(p['db1']),
        p['pW0'].T, _r2(p['pb0']), p['pW1'].T, _r2(p['pb1']),
    )
    return jnp.concatenate([s, d, pp], axis=-1).reshape(Bsz, N, 4)


# speedup vs baseline: 6.2508x; 1.7503x over previous
import functools

import jax
import jax.numpy as jnp
from jax import lax
from jax.experimental import pallas as pl
from jax.experimental.pallas import tpu as pltpu
from jax.experimental.pallas import tpu_sc as plsc

H = 64
H2 = 128
LAT = 128
F32 = jnp.float32

NC, NS = 2, 16
NW = NC * NS
CH = 128

N_NODES = 10000
E_EDGES = 160000
EP = 163840
NCHW = EP // (NW * CH)
NCHT = EP // (NS * CH)
NP = 10240
NPT = NP // NS

DEPTH_G = 2
DEPTH_S = 2


def _dot(a, b):
    return jnp.dot(a, b, preferred_element_type=F32)


def _gather_body(tbl_hbm, idx_hbm, out_hbm, idxv, buf, spt, semi, semo):
    c = lax.axis_index("c")
    s = lax.axis_index("s")
    pltpu.sync_copy(tbl_hbm.at[c, pl.ds(s * NPT, NPT)], spt.at[pl.ds(s * NPT, NPT)])
    pltpu.sync_copy(idx_hbm.at[c, pl.ds(s * NCHT, NCHT)], idxv)
    plsc.subcore_barrier()

    for d in range(DEPTH_G - 1):
        pltpu.async_copy(spt.at[idxv.at[d]], buf.at[d], semi.at[d])

    @pl.loop(0, NCHT)
    def _(j):
        slot = lax.rem(j, DEPTH_G)
        pltpu.make_async_copy(spt.at[idxv.at[j]], buf.at[slot], semi.at[slot]).wait()
        base = (s * NCHT + j) * CH
        pltpu.async_copy(buf.at[slot], out_hbm.at[c, pl.ds(base, CH)], semo.at[slot])
        jn = j + DEPTH_G - 1

        @pl.when(jn < NCHT)
        def _():
            ns = lax.rem(jn, DEPTH_G)

            @pl.when(j > 0)
            def _():
                pbase = (s * NCHT + j - 1) * CH
                pltpu.make_async_copy(
                    buf.at[ns], out_hbm.at[c, pl.ds(pbase, CH)], semo.at[ns]).wait()

            pltpu.async_copy(spt.at[idxv.at[jn]], buf.at[ns], semi.at[ns])

    for k in range(NCHT - DEPTH_G, NCHT):
        slot = k % DEPTH_G
        base = (s * NCHT + k) * CH
        pltpu.make_async_copy(buf.at[slot], out_hbm.at[c, pl.ds(base, CH)],
                              semo.at[slot]).wait()


@functools.lru_cache(maxsize=None)
def _gather_call():
    return pl.kernel(
        _gather_body,
        out_type=jax.ShapeDtypeStruct((NC, EP, H2), F32),
        mesh=plsc.VectorSubcoreMesh(core_axis_name="c", subcore_axis_name="s"),
        scratch_types=[
            pltpu.VMEM((NCHT, CH), jnp.int32),
            pltpu.VMEM((DEPTH_G, CH, H2), F32),
            pltpu.VMEM_SHARED((NP, H2), F32),
            pltpu.SemaphoreType.DMA((DEPTH_G,)),
            pltpu.SemaphoreType.DMA((DEPTH_G,)),
        ],
    )


def _scatter_body(e2_hbm, col_hbm, zero_hbm, out_hbm, colv, bufe, aggr, sem):
    c = lax.axis_index("c")
    s = lax.axis_index("s")
    wid = s * NC + c
    cbase = wid * NCHW
    pltpu.sync_copy(col_hbm.at[pl.ds(cbase, NCHW)], colv)
    pltpu.sync_copy(zero_hbm.at[pl.ds(s * NPT, NPT)], aggr.at[pl.ds(s * NPT, NPT)])
    plsc.subcore_barrier()

    for d in range(DEPTH_S - 1):
        pltpu.async_copy(e2_hbm.at[pl.ds((cbase + d) * CH, CH)], bufe.at[d], sem.at[d])

    @pl.loop(0, NCHW)
    def _(j):
        slot = lax.rem(j, DEPTH_S)
        pltpu.make_async_copy(
            e2_hbm.at[pl.ds((cbase + j) * CH, CH)], bufe.at[slot], sem.at[slot]
        ).wait()
        pltpu.sync_copy(bufe.at[slot], aggr.at[colv.at[j]], add=True)
        jn = j + DEPTH_S - 1

        @pl.when(jn < NCHW)
        def _():
            ns = lax.rem(jn, DEPTH_S)
            pltpu.async_copy(e2_hbm.at[pl.ds((cbase + jn) * CH, CH)], bufe.at[ns], sem.at[ns])

    plsc.subcore_barrier()
    pltpu.sync_copy(aggr.at[pl.ds(s * NPT, NPT)], out_hbm.at[c, pl.ds(s * NPT, NPT)])


@functools.lru_cache(maxsize=None)
def _scatter_call():
    return pl.kernel(
        _scatter_body,
        out_type=jax.ShapeDtypeStruct((NC, NP, H2), F32),
        mesh=plsc.VectorSubcoreMesh(core_axis_name="c", subcore_axis_name="s"),
        scratch_types=[
            pltpu.VMEM((NCHW, CH), jnp.int32),
            pltpu.VMEM((DEPTH_S, CH, H2), F32),
            pltpu.VMEM_SHARED((NP, H2), F32),
            pltpu.SemaphoreType.DMA((DEPTH_S,)),
        ],
    )


def _sc_gather(ab, idx3):
    return _gather_call()(ab, idx3)


def _sc_scatter(e2, col2d, zeros_nh):
    return _scatter_call()(e2, col2d, zeros_nh)


def _enc_body(x0_ref, x1_ref, w0t, b0, w1t, b1, wrt2, wct2, h_ref, ab_ref):
    def enc(xr):
        h = jnp.maximum(_dot(xr[0], w0t[...]) + b0[...], 0.0)
        return _dot(h, w1t[...]) + b1[...]

    h2 = jnp.concatenate([enc(x0_ref), enc(x1_ref)], axis=-1)
    h_ref[...] = h2
    ab_ref[0] = _dot(h2, wrt2[...])
    ab_ref[1] = _dot(h2, wct2[...])


def _edge_body(ga_ref, gb_ref, ea_ref, w0et2, b02, w1t2, b12, w2t2, b22,
               out_ref, *, be):
    e = ga_ref[0] + gb_ref[0] + _dot(ea_ref[...], w0et2[...]) + b02[...]
    e = jnp.maximum(e, 0.0)
    e = jnp.maximum(_dot(e, w1t2[...]) + b12[...], 0.0)
    e = _dot(e, w2t2[...]) + b22[...]
    gidx = lax.broadcasted_iota(jnp.int32, (be, 1), 0) + pl.program_id(0) * be
    out_ref[...] = jnp.where(gidx < E_EDGES, e, 0.0)


def _ln_half(hn, g, be):
    mu = jnp.mean(hn, axis=-1, keepdims=True)
    v = jnp.mean((hn - mu) ** 2, axis=-1, keepdims=True)
    return (hn - mu) / jnp.sqrt(v + 1e-5) * g + be


def _node_common(h_ref, ag0_ref, ag1_ref, w0ht2, w0at2, b02, w1t2, b12, g, be):
    h2 = h_ref[...]
    ag = ag0_ref[0] + ag1_ref[0]
    n = jnp.maximum(_dot(h2, w0ht2[...]) + _dot(ag, w0at2[...]) + b02[...], 0.0)
    n = _dot(n, w1t2[...]) + b12[...]
    hn = h2 + n
    return jnp.concatenate(
        [_ln_half(hn[:, :H], g[...], be[...]),
         _ln_half(hn[:, H:], g[...], be[...])], axis=-1)


def _node_proj_body(h_ref, ag0_ref, ag1_ref, w0ht2, w0at2, b02, w1t2, b12, g, be,
                    wrt2, wct2, h_out, ab_out):
    hnew = _node_common(h_ref, ag0_ref, ag1_ref, w0ht2, w0at2, b02, w1t2, b12, g, be)
    h_out[...] = hnew
    ab_out[0] = _dot(hnew, wrt2[...])
    ab_out[1] = _dot(hnew, wct2[...])


def _node_last_body(h_ref, ag0_ref, ag1_ref, w0ht2, w0at2, b02, w1t2, b12, g, be,
                    h_out):
    h_out[...] = _node_common(h_ref, ag0_ref, ag1_ref, w0ht2, w0at2, b02, w1t2,
                              b12, g, be)


def _sig(x):
    return jax.nn.sigmoid(x)


def _lstm_cell(gates, c):
    i, f, g, o = jnp.split(gates, 4, axis=-1)
    c = _sig(f) * c + _sig(i) * jnp.tanh(g)
    return _sig(o) * jnp.tanh(c), c


def _head(h, w0t, b0, w1t, b1):
    return _dot(jnp.maximum(_dot(h, w0t[...]) + b0[...], 0.0), w1t[...]) + b1[...]


def _lstm_head_body(h2_ref,
                    wih1t, whh1t, bb1, wih2t, whh2t, bb2,
                    sw0t, sb0, sw1t, sb1, dw0t, db0, dw1t, db1,
                    pw0t, pb0, pw1t, pb1,
                    s_out, d_out, p_out):
    x0 = h2_ref[:, :H]
    x1 = h2_ref[:, H:]
    g0 = _dot(x0, wih1t[...]) + bb1[...]
    y0, c = _lstm_cell(g0, 0.0)
    g1 = _dot(x1, wih1t[...]) + _dot(y0, whh1t[...]) + bb1[...]
    y1, c = _lstm_cell(g1, c)
    g0 = _dot(y0, wih2t[...]) + bb2[...]
    z0, c = _lstm_cell(g0, 0.0)
    g1 = _dot(y1, wih2t[...]) + _dot(z0, whh2t[...]) + bb2[...]
    z1, c = _lstm_cell(g1, c)
    s_out[...] = _head(z1, sw0t, sb0, sw1t, sb1)
    d = _head(z1, dw0t, db0, dw1t, db1)
    nrm = jnp.sqrt(jnp.sum(d * d, axis=-1, keepdims=True))
    d_out[...] = d / jnp.maximum(nrm, 1e-12)
    p_out[...] = _head(z1, pw0t, pb0, pw1t, pb1)


def _full_spec(shape):
    return pl.BlockSpec(shape, lambda i: tuple(0 for _ in shape))


def _row_spec(bn, cols):
    return pl.BlockSpec((bn, cols), lambda i: (i, 0))


def _tc_call(body, grid, in_specs, out_specs, out_shapes):
    return pl.pallas_call(
        body,
        grid=(grid,),
        in_specs=in_specs,
        out_specs=out_specs,
        out_shape=out_shapes,
    )


def _r2(v):
    return v.reshape(1, -1)


def _bd(wt):
    k, m = wt.shape
    z = jnp.zeros((k, m), wt.dtype)
    return jnp.concatenate(
        [jnp.concatenate([wt, z], axis=1), jnp.concatenate([z, wt], axis=1)], axis=0)


def _p2(v):
    return jnp.concatenate([v, v]).reshape(1, -1)


def kernel(x, edge_index, edge_attr, params):
    Bsz, T, N, INP = x.shape
    E = edge_index.shape[1]
    p = params
    pad = EP - E
    row2d = jnp.pad(edge_index[0], (0, pad)).reshape(EP // CH, CH)
    col2d = jnp.pad(edge_index[1], (0, pad)).reshape(EP // CH, CH)
    idx3 = jnp.stack([row2d, col2d])
    zeros_nh = jnp.zeros((NP, H2), F32)

    xsp = jnp.zeros((T, NP, INP), F32).at[:, :N].set(x[0])
    lyr0 = p['layers'][0]

    BN = 2048
    nblk = NP // BN
    BNL = 2000

    enc_in = [
        pl.BlockSpec((1, BN, INP), lambda i: (0, i, 0)),
        pl.BlockSpec((1, BN, INP), lambda i: (1, i, 0)),
        _full_spec((INP, H)), _full_spec((1, H)),
        _full_spec((H, H)), _full_spec((1, H)),
        _full_spec((H2, H2)), _full_spec((H2, H2)),
    ]
    enc_out = [_row_spec(BN, H2), pl.BlockSpec((2, BN, H2), lambda i: (0, i, 0))]
    h2, ab = _tc_call(
        _enc_body, nblk, enc_in, enc_out,
        [jax.ShapeDtypeStruct((NP, H2), F32),
         jax.ShapeDtypeStruct((2, NP, H2), F32)],
    )(xsp, xsp, p['encW0'].T, _r2(p['encb0']), p['encW1'].T, _r2(p['encb1']),
      _bd(lyr0['eW0'][:, :H].T), _bd(lyr0['eW0'][:, H:2 * H].T))

    EDIM = edge_attr.shape[1]
    ea8 = jnp.pad(edge_attr, ((0, pad), (0, 8 - EDIM)))

    BE = 4096
    eblk = EP // BE

    def edge_mlp(gab, lyr):
        w0et = jnp.pad(lyr['eW0'][:, 2 * H:].T, ((0, 8 - EDIM), (0, 0)))
        w0et2 = jnp.concatenate([w0et, w0et], axis=1)
        ga_spec = pl.BlockSpec((1, BE, H2), lambda i: (0, i, 0))
        gb_spec = pl.BlockSpec((1, BE, H2), lambda i: (1, i, 0))
        specs = [
            ga_spec, gb_spec, _row_spec(BE, 8),
            _full_spec((8, H2)), _full_spec((1, H2)),
            _full_spec((H2, H2)), _full_spec((1, H2)),
            _full_spec((H2, H2)), _full_spec((1, H2)),
        ]
        return _tc_call(
            functools.partial(_edge_body, be=BE), eblk, specs, _row_spec(BE, H2),
            jax.ShapeDtypeStruct((EP, H2), F32),
        )(gab, gab, ea8, w0et2, _p2(lyr['eb0']), _bd(lyr['eW1'].T), _p2(lyr['eb1']),
          _bd(lyr['eW2'].T), _p2(lyr['eb2']))

    def node_update(h2c, parts, lyr, nxt):
        w_common = (_bd(lyr['nW0'][:, :H].T), _bd(lyr['nW0'][:, H:].T), _p2(lyr['nb0']),
                    _bd(lyr['nW1'].T), _p2(lyr['nb1']), _r2(lyr['g']), _r2(lyr['be']))
        ag0_spec = pl.BlockSpec((1, BN, H2), lambda i: (0, i, 0))
        ag1_spec = pl.BlockSpec((1, BN, H2), lambda i: (1, i, 0))
        specs_common = [
            _row_spec(BN, H2), ag0_spec, ag1_spec,
            _full_spec((H2, H2)), _full_spec((H2, H2)), _full_spec((1, H2)),
            _full_spec((H2, H2)), _full_spec((1, H2)),
            _full_spec((1, H)), _full_spec((1, H)),
        ]
        if nxt is None:
            return _tc_call(
                _node_last_body, nblk, specs_common, _row_spec(BN, H2),
                jax.ShapeDtypeStruct((NP, H2), F32),
            )(h2c, parts, parts, *w_common)
        specs = specs_common + [_full_spec((H2, H2)), _full_spec((H2, H2))]
        out_specs = [_row_spec(BN, H2), pl.BlockSpec((2, BN, H2), lambda i: (0, i, 0))]
        return _tc_call(
            _node_proj_body, nblk, specs, out_specs,
            [jax.ShapeDtypeStruct((NP, H2), F32),
             jax.ShapeDtypeStruct((2, NP, H2), F32)],
        )(h2c, parts, parts, *w_common,
          _bd(nxt['eW0'][:, :H].T), _bd(nxt['eW0'][:, H:2 * H].T))

    NL = len(p['layers'])
    for li, lyr in enumerate(p['layers']):
        nxt = p['layers'][li + 1] if li + 1 < NL else None
        gab = _sc_gather(ab, idx3)
        e2 = edge_mlp(gab, lyr)
        parts = _sc_scatter(e2, col2d, zeros_nh)
        if nxt is None:
            h2 = node_update(h2, parts, lyr, None)
        else:
            h2, ab = node_update(h2, parts, lyr, nxt)

    lp1, lp2 = p['lstm']
    specs = [
        _row_spec(BNL, H2),
        _full_spec((H, 4 * LAT)), _full_spec((LAT, 4 * LAT)), _full_spec((1, 4 * LAT)),
        _full_spec((LAT, 4 * LAT)), _full_spec((LAT, 4 * LAT)), _full_spec((1, 4 * LAT)),
        _full_spec((LAT, H // 2)), _full_spec((1, H // 2)), _full_spec((H // 2, 1)), _full_spec((1, 1)),
        _full_spec((LAT, H // 2)), _full_spec((1, H // 2)), _full_spec((H // 2, 2)), _full_spec((1, 2)),
        _full_spec((LAT, H // 2)), _full_spec((1, H // 2)), _full_spec((H // 2, 1)), _full_spec((1, 1)),
    ]
    out_specs = [_row_spec(BNL, 1), _row_spec(BNL, 2), _row_spec(BNL, 1)]
    out_shapes = [jax.ShapeDtypeStruct((N, 1), F32),
                  jax.ShapeDtypeStruct((N, 2), F32),
                  jax.ShapeDtypeStruct((N, 1), F32)]
    s, d, pp = _tc_call(_lstm_head_body, N // BNL, specs, out_specs, out_shapes)(
        h2,
        lp1['Wih'].T, lp1['Whh'].T, _r2(lp1['bih'] + lp1['bhh']),
        lp2['Wih'].T, lp2['Whh'].T, _r2(lp2['bih'] + lp2['bhh']),
        p['sW0'].T, _r2(p['sb0']), p['sW1'].T, _r2(p['sb1']),
        p['dW0'].T, _r2(p['db0']), p['dW1'].T, _r2(p['db1']),
        p['pW0'].T, _r2(p['pb0']), p['pW1'].T, _r2(p['pb1']),
    )
    return jnp.concatenate([s, d, pp], axis=-1).reshape(Bsz, N, 4)


# speedup vs baseline: 6.4549x; 1.0326x over previous
import functools

import jax
import jax.numpy as jnp
from jax import lax
from jax.experimental import pallas as pl
from jax.experimental.pallas import tpu as pltpu
from jax.experimental.pallas import tpu_sc as plsc

H = 64
H2 = 128
LAT = 128
F32 = jnp.float32

NC, NS = 2, 16
NW = NC * NS
CH = 128

N_NODES = 10000
E_EDGES = 160000
EP = 163840
NCHW = EP // (NW * CH)
NCHT = EP // (NS * CH)
NP = 10240
NPT = NP // NS
EH = EP // 2
NCHT_G = EH // (NS * CH)
NCHW_S = EH // (NW * CH)

DEPTH_G = 2
DEPTH_S = 2


def _dot(a, b):
    return jnp.dot(a, b, preferred_element_type=F32)


def _gather_body(tbl_hbm, idx_hbm, out_hbm, idxv, buf, spt, semi, semo):
    c = lax.axis_index("c")
    s = lax.axis_index("s")
    pltpu.sync_copy(tbl_hbm.at[c, pl.ds(s * NPT, NPT)], spt.at[pl.ds(s * NPT, NPT)])
    pltpu.sync_copy(idx_hbm.at[c, pl.ds(s * NCHT_G, NCHT_G)], idxv)
    plsc.subcore_barrier()

    for d in range(DEPTH_G - 1):
        pltpu.async_copy(spt.at[idxv.at[d]], buf.at[d], semi.at[d])

    @pl.loop(0, NCHT_G)
    def _(j):
        slot = lax.rem(j, DEPTH_G)
        pltpu.make_async_copy(spt.at[idxv.at[j]], buf.at[slot], semi.at[slot]).wait()
        base = (s * NCHT_G + j) * CH
        pltpu.async_copy(buf.at[slot], out_hbm.at[c, pl.ds(base, CH)], semo.at[slot])
        jn = j + DEPTH_G - 1

        @pl.when(jn < NCHT_G)
        def _():
            ns = lax.rem(jn, DEPTH_G)

            @pl.when(j > 0)
            def _():
                pbase = (s * NCHT_G + j - 1) * CH
                pltpu.make_async_copy(
                    buf.at[ns], out_hbm.at[c, pl.ds(pbase, CH)], semo.at[ns]).wait()

            pltpu.async_copy(spt.at[idxv.at[jn]], buf.at[ns], semi.at[ns])

    for k in range(NCHT_G - DEPTH_G, NCHT_G):
        slot = k % DEPTH_G
        base = (s * NCHT_G + k) * CH
        pltpu.make_async_copy(buf.at[slot], out_hbm.at[c, pl.ds(base, CH)],
                              semo.at[slot]).wait()


@functools.lru_cache(maxsize=None)
def _gather_call():
    return pl.kernel(
        _gather_body,
        out_type=jax.ShapeDtypeStruct((NC, EH, H2), F32),
        mesh=plsc.VectorSubcoreMesh(core_axis_name="c", subcore_axis_name="s"),
        scratch_types=[
            pltpu.VMEM((NCHT_G, CH), jnp.int32),
            pltpu.VMEM((DEPTH_G, CH, H2), F32),
            pltpu.VMEM_SHARED((NP, H2), F32),
            pltpu.SemaphoreType.DMA((DEPTH_G,)),
            pltpu.SemaphoreType.DMA((DEPTH_G,)),
        ],
    )


def _scatter_body(e2_hbm, col_hbm, zero_hbm, out_hbm, colv, bufe, aggr, sem):
    c = lax.axis_index("c")
    s = lax.axis_index("s")
    wid = s * NC + c
    cbase = wid * NCHW_S
    pltpu.sync_copy(col_hbm.at[wid], colv)
    pltpu.sync_copy(zero_hbm.at[pl.ds(s * NPT, NPT)], aggr.at[pl.ds(s * NPT, NPT)])
    plsc.subcore_barrier()

    for d in range(DEPTH_S - 1):
        pltpu.async_copy(e2_hbm.at[pl.ds((cbase + d) * CH, CH)], bufe.at[d], sem.at[d])

    @pl.loop(0, NCHW_S)
    def _(j):
        slot = lax.rem(j, DEPTH_S)
        pltpu.make_async_copy(
            e2_hbm.at[pl.ds((cbase + j) * CH, CH)], bufe.at[slot], sem.at[slot]
        ).wait()
        pltpu.sync_copy(bufe.at[slot], aggr.at[colv.at[j]], add=True)
        jn = j + DEPTH_S - 1

        @pl.when(jn < NCHW_S)
        def _():
            ns = lax.rem(jn, DEPTH_S)
            pltpu.async_copy(e2_hbm.at[pl.ds((cbase + jn) * CH, CH)], bufe.at[ns], sem.at[ns])

    plsc.subcore_barrier()
    pltpu.sync_copy(aggr.at[pl.ds(s * NPT, NPT)], out_hbm.at[c, pl.ds(s * NPT, NPT)])


@functools.lru_cache(maxsize=None)
def _scatter_call():
    return pl.kernel(
        _scatter_body,
        out_type=jax.ShapeDtypeStruct((NC, NP, H2), F32),
        mesh=plsc.VectorSubcoreMesh(core_axis_name="c", subcore_axis_name="s"),
        scratch_types=[
            pltpu.VMEM((NCHW_S, CH), jnp.int32),
            pltpu.VMEM((DEPTH_S, CH, H2), F32),
            pltpu.VMEM_SHARED((NP, H2), F32),
            pltpu.SemaphoreType.DMA((DEPTH_S,)),
        ],
    )


def _sc_gather(ab, idx3):
    return _gather_call()(ab, idx3)


def _sc_scatter(e2, col2d, zeros_nh):
    return _scatter_call()(e2, col2d, zeros_nh)


def _enc_body(x0_ref, x1_ref, w0t, b0, w1t, b1, wrt2, wct2, h_ref, ab_ref):
    def enc(xr):
        h = jnp.maximum(_dot(xr[0], w0t[...]) + b0[...], 0.0)
        return _dot(h, w1t[...]) + b1[...]

    h2 = jnp.concatenate([enc(x0_ref), enc(x1_ref)], axis=-1)
    h_ref[...] = h2
    ab_ref[0] = _dot(h2, wrt2[...])
    ab_ref[1] = _dot(h2, wct2[...])


def _edge_body(ga_ref, gb_ref, ea_ref, w0et2, b02, w1t2, b12, w2t2, b22,
               out_ref, *, be, eoff):
    e = ga_ref[0] + gb_ref[0] + _dot(ea_ref[...], w0et2[...]) + b02[...]
    e = jnp.maximum(e, 0.0)
    e = jnp.maximum(_dot(e, w1t2[...]) + b12[...], 0.0)
    e = _dot(e, w2t2[...]) + b22[...]
    gidx = lax.broadcasted_iota(jnp.int32, (be, 1), 0) + pl.program_id(0) * be + eoff
    out_ref[...] = jnp.where(gidx < E_EDGES, e, 0.0)


def _ln_half(hn, g, be):
    mu = jnp.mean(hn, axis=-1, keepdims=True)
    v = jnp.mean((hn - mu) ** 2, axis=-1, keepdims=True)
    return (hn - mu) / jnp.sqrt(v + 1e-5) * g + be


def _node_common(h_ref, ag0_ref, ag1_ref, ag2_ref, ag3_ref, w0ht2, w0at2, b02,
                 w1t2, b12, g, be):
    h2 = h_ref[...]
    ag = (ag0_ref[0] + ag1_ref[0]) + (ag2_ref[0] + ag3_ref[0])
    n = jnp.maximum(_dot(h2, w0ht2[...]) + _dot(ag, w0at2[...]) + b02[...], 0.0)
    n = _dot(n, w1t2[...]) + b12[...]
    hn = h2 + n
    return jnp.concatenate(
        [_ln_half(hn[:, :H], g[...], be[...]),
         _ln_half(hn[:, H:], g[...], be[...])], axis=-1)


def _node_proj_body(h_ref, ag0_ref, ag1_ref, ag2_ref, ag3_ref, w0ht2, w0at2, b02,
                    w1t2, b12, g, be, wrt2, wct2, h_out, ab_out):
    hnew = _node_common(h_ref, ag0_ref, ag1_ref, ag2_ref, ag3_ref, w0ht2, w0at2,
                        b02, w1t2, b12, g, be)
    h_out[...] = hnew
    ab_out[0] = _dot(hnew, wrt2[...])
    ab_out[1] = _dot(hnew, wct2[...])


def _node_last_body(h_ref, ag0_ref, ag1_ref, ag2_ref, ag3_ref, w0ht2, w0at2, b02,
                    w1t2, b12, g, be, h_out):
    h_out[...] = _node_common(h_ref, ag0_ref, ag1_ref, ag2_ref, ag3_ref, w0ht2,
                              w0at2, b02, w1t2, b12, g, be)


def _sig(x):
    return jax.nn.sigmoid(x)


def _lstm_cell(gates, c):
    i, f, g, o = jnp.split(gates, 4, axis=-1)
    c = _sig(f) * c + _sig(i) * jnp.tanh(g)
    return _sig(o) * jnp.tanh(c), c


def _head(h, w0t, b0, w1t, b1):
    return _dot(jnp.maximum(_dot(h, w0t[...]) + b0[...], 0.0), w1t[...]) + b1[...]


def _lstm_head_body(h2_ref,
                    wih1t, whh1t, bb1, wih2t, whh2t, bb2,
                    sw0t, sb0, sw1t, sb1, dw0t, db0, dw1t, db1,
                    pw0t, pb0, pw1t, pb1,
                    s_out, d_out, p_out):
    x0 = h2_ref[:, :H]
    x1 = h2_ref[:, H:]
    g0 = _dot(x0, wih1t[...]) + bb1[...]
    y0, c = _lstm_cell(g0, 0.0)
    g1 = _dot(x1, wih1t[...]) + _dot(y0, whh1t[...]) + bb1[...]
    y1, c = _lstm_cell(g1, c)
    g0 = _dot(y0, wih2t[...]) + bb2[...]
    z0, c = _lstm_cell(g0, 0.0)
    g1 = _dot(y1, wih2t[...]) + _dot(z0, whh2t[...]) + bb2[...]
    z1, c = _lstm_cell(g1, c)
    s_out[...] = _head(z1, sw0t, sb0, sw1t, sb1)
    d = _head(z1, dw0t, db0, dw1t, db1)
    nrm = jnp.sqrt(jnp.sum(d * d, axis=-1, keepdims=True))
    d_out[...] = d / jnp.maximum(nrm, 1e-12)
    p_out[...] = _head(z1, pw0t, pb0, pw1t, pb1)


def _full_spec(shape):
    return pl.BlockSpec(shape, lambda i: tuple(0 for _ in shape))


def _row_spec(bn, cols):
    return pl.BlockSpec((bn, cols), lambda i: (i, 0))


def _tc_call(body, grid, in_specs, out_specs, out_shapes):
    return pl.pallas_call(
        body,
        grid=(grid,),
        in_specs=in_specs,
        out_specs=out_specs,
        out_shape=out_shapes,
    )


def _r2(v):
    return v.reshape(1, -1)


def _bd(wt):
    k, m = wt.shape
    z = jnp.zeros((k, m), wt.dtype)
    return jnp.concatenate(
        [jnp.concatenate([wt, z], axis=1), jnp.concatenate([z, wt], axis=1)], axis=0)


def _p2(v):
    return jnp.concatenate([v, v]).reshape(1, -1)


def kernel(x, edge_index, edge_attr, params):
    Bsz, T, N, INP = x.shape
    E = edge_index.shape[1]
    p = params
    pad = EP - E
    row2d = jnp.pad(edge_index[0], (0, pad)).reshape(EP // CH, CH)
    col2d = jnp.pad(edge_index[1], (0, pad)).reshape(EP // CH, CH)
    idx3 = jnp.stack([row2d, col2d])
    zeros_nh = jnp.zeros((NP, H2), F32)

    xsp = jnp.zeros((T, NP, INP), F32).at[:, :N].set(x[0])
    lyr0 = p['layers'][0]

    BN = 2048
    nblk = NP // BN
    BNL = 2000

    enc_in = [
        pl.BlockSpec((1, BN, INP), lambda i: (0, i, 0)),
        pl.BlockSpec((1, BN, INP), lambda i: (1, i, 0)),
        _full_spec((INP, H)), _full_spec((1, H)),
        _full_spec((H, H)), _full_spec((1, H)),
        _full_spec((H2, H2)), _full_spec((H2, H2)),
    ]
    enc_out = [_row_spec(BN, H2), pl.BlockSpec((2, BN, H2), lambda i: (0, i, 0))]
    h2, ab = _tc_call(
        _enc_body, nblk, enc_in, enc_out,
        [jax.ShapeDtypeStruct((NP, H2), F32),
         jax.ShapeDtypeStruct((2, NP, H2), F32)],
    )(xsp, xsp, p['encW0'].T, _r2(p['encb0']), p['encW1'].T, _r2(p['encb1']),
      _bd(lyr0['eW0'][:, :H].T), _bd(lyr0['eW0'][:, H:2 * H].T))

    EDIM = edge_attr.shape[1]
    ea8 = jnp.pad(edge_attr, ((0, pad), (0, 8 - EDIM)))

    BE = 4096
    eblk = EH // BE

    def edge_mlp(gab, lyr, half):
        w0et = jnp.pad(lyr['eW0'][:, 2 * H:].T, ((0, 8 - EDIM), (0, 0)))
        w0et2 = jnp.concatenate([w0et, w0et], axis=1)
        hb = half * (EH // BE)
        ga_spec = pl.BlockSpec((1, BE, H2), lambda i: (0, i, 0))
        gb_spec = pl.BlockSpec((1, BE, H2), lambda i: (1, i, 0))
        ea_spec = pl.BlockSpec((BE, 8), lambda i: (i + hb, 0))
        specs = [
            ga_spec, gb_spec, ea_spec,
            _full_spec((8, H2)), _full_spec((1, H2)),
            _full_spec((H2, H2)), _full_spec((1, H2)),
            _full_spec((H2, H2)), _full_spec((1, H2)),
        ]
        return _tc_call(
            functools.partial(_edge_body, be=BE, eoff=half * EH), eblk, specs,
            _row_spec(BE, H2),
            jax.ShapeDtypeStruct((EH, H2), F32),
        )(gab, gab, ea8, w0et2, _p2(lyr['eb0']), _bd(lyr['eW1'].T), _p2(lyr['eb1']),
          _bd(lyr['eW2'].T), _p2(lyr['eb2']))

    def node_update(h2c, parts0, parts1, lyr, nxt):
        w_common = (_bd(lyr['nW0'][:, :H].T), _bd(lyr['nW0'][:, H:].T), _p2(lyr['nb0']),
                    _bd(lyr['nW1'].T), _p2(lyr['nb1']), _r2(lyr['g']), _r2(lyr['be']))
        ag0_spec = pl.BlockSpec((1, BN, H2), lambda i: (0, i, 0))
        ag1_spec = pl.BlockSpec((1, BN, H2), lambda i: (1, i, 0))
        specs_common = [
            _row_spec(BN, H2), ag0_spec, ag1_spec, ag0_spec, ag1_spec,
            _full_spec((H2, H2)), _full_spec((H2, H2)), _full_spec((1, H2)),
            _full_spec((H2, H2)), _full_spec((1, H2)),
            _full_spec((1, H)), _full_spec((1, H)),
        ]
        if nxt is None:
            return _tc_call(
                _node_last_body, nblk, specs_common, _row_spec(BN, H2),
                jax.ShapeDtypeStruct((NP, H2), F32),
            )(h2c, parts0, parts0, parts1, parts1, *w_common)
        specs = specs_common + [_full_spec((H2, H2)), _full_spec((H2, H2))]
        out_specs = [_row_spec(BN, H2), pl.BlockSpec((2, BN, H2), lambda i: (0, i, 0))]
        return _tc_call(
            _node_proj_body, nblk, specs, out_specs,
            [jax.ShapeDtypeStruct((NP, H2), F32),
             jax.ShapeDtypeStruct((2, NP, H2), F32)],
        )(h2c, parts0, parts0, parts1, parts1, *w_common,
          _bd(nxt['eW0'][:, :H].T), _bd(nxt['eW0'][:, H:2 * H].T))

    EHC = EH // CH
    idxA = idx3[:, :EHC]
    idxB = idx3[:, EHC:]
    colA = col2d[:EHC].reshape(NW, NCHW_S, CH)
    colB = col2d[EHC:].reshape(NW, NCHW_S, CH)

    NL = len(p['layers'])
    for li, lyr in enumerate(p['layers']):
        nxt = p['layers'][li + 1] if li + 1 < NL else None
        gab0 = _sc_gather(ab, idxA)
        gab1 = _sc_gather(ab, idxB)
        e0 = edge_mlp(gab0, lyr, 0)
        e1 = edge_mlp(gab1, lyr, 1)
        parts0 = _sc_scatter(e0, colA, zeros_nh)
        parts1 = _sc_scatter(e1, colB, zeros_nh)
        if nxt is None:
            h2 = node_update(h2, parts0, parts1, lyr, None)
        else:
            h2, ab = node_update(h2, parts0, parts1, lyr, nxt)

    lp1, lp2 = p['lstm']
    specs = [
        _row_spec(BNL, H2),
        _full_spec((H, 4 * LAT)), _full_spec((LAT, 4 * LAT)), _full_spec((1, 4 * LAT)),
        _full_spec((LAT, 4 * LAT)), _full_spec((LAT, 4 * LAT)), _full_spec((1, 4 * LAT)),
        _full_spec((LAT, H // 2)), _full_spec((1, H // 2)), _full_spec((H // 2, 1)), _full_spec((1, 1)),
        _full_spec((LAT, H // 2)), _full_spec((1, H // 2)), _full_spec((H // 2, 2)), _full_spec((1, 2)),
        _full_spec((LAT, H // 2)), _full_spec((1, H // 2)), _full_spec((H // 2, 1)), _full_spec((1, 1)),
    ]
    out_specs = [_row_spec(BNL, 1), _row_spec(BNL, 2), _row_spec(BNL, 1)]
    out_shapes = [jax.ShapeDtypeStruct((N, 1), F32),
                  jax.ShapeDtypeStruct((N, 2), F32),
                  jax.ShapeDtypeStruct((N, 1), F32)]
    s, d, pp = _tc_call(_lstm_head_body, N // BNL, specs, out_specs, out_shapes)(
        h2,
        lp1['Wih'].T, lp1['Whh'].T, _r2(lp1['bih'] + lp1['bhh']),
        lp2['Wih'].T, lp2['Whh'].T, _r2(lp2['bih'] + lp2['bhh']),
        p['sW0'].T, _r2(p['sb0']), p['sW1'].T, _r2(p['sb1']),
        p['dW0'].T, _r2(p['db0']), p['dW1'].T, _r2(p['db1']),
        p['pW0'].T, _r2(p['pb0']), p['pW1'].T, _r2(p['pb1']),
    )
    return jnp.concatenate([s, d, pp], axis=-1).reshape(Bsz, N, 4)


# speedup vs baseline: 6.4794x; 1.0038x over previous
import functools

import jax
import jax.numpy as jnp
from jax import lax
from jax.experimental import pallas as pl
from jax.experimental.pallas import tpu as pltpu
from jax.experimental.pallas import tpu_sc as plsc

H = 64
H2 = 128
LAT = 128
F32 = jnp.float32

NC, NS = 2, 16
NW = NC * NS
CH = 128

N_NODES = 10000
E_EDGES = 160000
EP = 163840
NCHW = EP // (NW * CH)
NCHT = EP // (NS * CH)
NP = 10240
NPT = NP // NS
EH = EP // 2
NCHT_G = EH // (NS * CH)
NCHW_S = EH // (NW * CH)

DEPTH_G = 2
DEPTH_S = 2


def _dot(a, b):
    return jnp.dot(a, b, preferred_element_type=F32)


def _gather_body(tbl_hbm, idx_hbm, out_hbm, idxv, buf, spt, semi, semo):
    c = lax.axis_index("c")
    s = lax.axis_index("s")
    stg = pltpu.async_copy(tbl_hbm.at[c, pl.ds(s * NPT, NPT)],
                           spt.at[pl.ds(s * NPT, NPT)], semi.at[0])
    pltpu.sync_copy(idx_hbm.at[c, pl.ds(s * NCHT_G, NCHT_G)], idxv)
    stg.wait()
    plsc.subcore_barrier()

    for d in range(DEPTH_G - 1):
        pltpu.async_copy(spt.at[idxv.at[d]], buf.at[d], semi.at[d])

    @pl.loop(0, NCHT_G)
    def _(j):
        slot = lax.rem(j, DEPTH_G)
        pltpu.make_async_copy(spt.at[idxv.at[j]], buf.at[slot], semi.at[slot]).wait()
        base = (s * NCHT_G + j) * CH
        pltpu.async_copy(buf.at[slot], out_hbm.at[c, pl.ds(base, CH)], semo.at[slot])
        jn = j + DEPTH_G - 1

        @pl.when(jn < NCHT_G)
        def _():
            ns = lax.rem(jn, DEPTH_G)

            @pl.when(j > 0)
            def _():
                pbase = (s * NCHT_G + j - 1) * CH
                pltpu.make_async_copy(
                    buf.at[ns], out_hbm.at[c, pl.ds(pbase, CH)], semo.at[ns]).wait()

            pltpu.async_copy(spt.at[idxv.at[jn]], buf.at[ns], semi.at[ns])

    for k in range(NCHT_G - DEPTH_G, NCHT_G):
        slot = k % DEPTH_G
        base = (s * NCHT_G + k) * CH
        pltpu.make_async_copy(buf.at[slot], out_hbm.at[c, pl.ds(base, CH)],
                              semo.at[slot]).wait()


@functools.lru_cache(maxsize=None)
def _gather_call():
    return pl.kernel(
        _gather_body,
        out_type=jax.ShapeDtypeStruct((NC, EH, H2), F32),
        mesh=plsc.VectorSubcoreMesh(core_axis_name="c", subcore_axis_name="s"),
        scratch_types=[
            pltpu.VMEM((NCHT_G, CH), jnp.int32),
            pltpu.VMEM((DEPTH_G, CH, H2), F32),
            pltpu.VMEM_SHARED((NP, H2), F32),
            pltpu.SemaphoreType.DMA((DEPTH_G,)),
            pltpu.SemaphoreType.DMA((DEPTH_G,)),
        ],
    )


def _scatter_body(e2_hbm, col_hbm, zero_hbm, out_hbm, colv, bufe, aggr, sem):
    c = lax.axis_index("c")
    s = lax.axis_index("s")
    wid = s * NC + c
    cbase = wid * NCHW_S
    stg = pltpu.async_copy(zero_hbm.at[pl.ds(s * NPT, NPT)],
                           aggr.at[pl.ds(s * NPT, NPT)], sem.at[0])
    pltpu.sync_copy(col_hbm.at[wid], colv)
    stg.wait()
    plsc.subcore_barrier()

    for d in range(DEPTH_S - 1):
        pltpu.async_copy(e2_hbm.at[pl.ds((cbase + d) * CH, CH)], bufe.at[d], sem.at[d])

    @pl.loop(0, NCHW_S)
    def _(j):
        slot = lax.rem(j, DEPTH_S)
        pltpu.make_async_copy(
            e2_hbm.at[pl.ds((cbase + j) * CH, CH)], bufe.at[slot], sem.at[slot]
        ).wait()
        pltpu.sync_copy(bufe.at[slot], aggr.at[colv.at[j]], add=True)
        jn = j + DEPTH_S - 1

        @pl.when(jn < NCHW_S)
        def _():
            ns = lax.rem(jn, DEPTH_S)
            pltpu.async_copy(e2_hbm.at[pl.ds((cbase + jn) * CH, CH)], bufe.at[ns], sem.at[ns])

    plsc.subcore_barrier()
    pltpu.sync_copy(aggr.at[pl.ds(s * NPT, NPT)], out_hbm.at[c, pl.ds(s * NPT, NPT)])


@functools.lru_cache(maxsize=None)
def _scatter_call():
    return pl.kernel(
        _scatter_body,
        out_type=jax.ShapeDtypeStruct((NC, NP, H2), F32),
        mesh=plsc.VectorSubcoreMesh(core_axis_name="c", subcore_axis_name="s"),
        scratch_types=[
            pltpu.VMEM((NCHW_S, CH), jnp.int32),
            pltpu.VMEM((DEPTH_S, CH, H2), F32),
            pltpu.VMEM_SHARED((NP, H2), F32),
            pltpu.SemaphoreType.DMA((DEPTH_S,)),
        ],
    )


def _sc_gather(ab, idx3):
    return _gather_call()(ab, idx3)


def _sc_scatter(e2, col2d, zeros_nh):
    return _scatter_call()(e2, col2d, zeros_nh)


def _enc_body(x0_ref, x1_ref, w0t, b0, w1t, b1, wrt2, wct2, h_ref, ab_ref):
    def enc(xr):
        h = jnp.maximum(_dot(xr[0], w0t[...]) + b0[...], 0.0)
        return _dot(h, w1t[...]) + b1[...]

    h2 = jnp.concatenate([enc(x0_ref), enc(x1_ref)], axis=-1)
    h_ref[...] = h2
    ab_ref[0] = _dot(h2, wrt2[...])
    ab_ref[1] = _dot(h2, wct2[...])


def _edge_body(ga_ref, gb_ref, ea_ref, w0et2, b02, w1t2, b12, w2t2, b22,
               out_ref, *, be, eoff):
    e = ga_ref[0] + gb_ref[0] + _dot(ea_ref[...], w0et2[...]) + b02[...]
    e = jnp.maximum(e, 0.0)
    e = jnp.maximum(_dot(e, w1t2[...]) + b12[...], 0.0)
    e = _dot(e, w2t2[...]) + b22[...]
    gidx = lax.broadcasted_iota(jnp.int32, (be, 1), 0) + pl.program_id(0) * be + eoff
    out_ref[...] = jnp.where(gidx < E_EDGES, e, 0.0)


def _ln_half(hn, g, be):
    mu = jnp.mean(hn, axis=-1, keepdims=True)
    v = jnp.mean((hn - mu) ** 2, axis=-1, keepdims=True)
    return (hn - mu) / jnp.sqrt(v + 1e-5) * g + be


def _node_common(h_ref, ag0_ref, ag1_ref, ag2_ref, ag3_ref, w0ht2, w0at2, b02,
                 w1t2, b12, g, be):
    h2 = h_ref[...]
    ag = (ag0_ref[0] + ag1_ref[0]) + (ag2_ref[0] + ag3_ref[0])
    n = jnp.maximum(_dot(h2, w0ht2[...]) + _dot(ag, w0at2[...]) + b02[...], 0.0)
    n = _dot(n, w1t2[...]) + b12[...]
    hn = h2 + n
    return jnp.concatenate(
        [_ln_half(hn[:, :H], g[...], be[...]),
         _ln_half(hn[:, H:], g[...], be[...])], axis=-1)


def _node_proj_body(h_ref, ag0_ref, ag1_ref, ag2_ref, ag3_ref, w0ht2, w0at2, b02,
                    w1t2, b12, g, be, wrt2, wct2, h_out, ab_out):
    hnew = _node_common(h_ref, ag0_ref, ag1_ref, ag2_ref, ag3_ref, w0ht2, w0at2,
                        b02, w1t2, b12, g, be)
    h_out[...] = hnew
    ab_out[0] = _dot(hnew, wrt2[...])
    ab_out[1] = _dot(hnew, wct2[...])


def _node_last_body(h_ref, ag0_ref, ag1_ref, ag2_ref, ag3_ref, w0ht2, w0at2, b02,
                    w1t2, b12, g, be, h_out):
    h_out[...] = _node_common(h_ref, ag0_ref, ag1_ref, ag2_ref, ag3_ref, w0ht2,
                              w0at2, b02, w1t2, b12, g, be)


def _sig(x):
    return jax.nn.sigmoid(x)


def _lstm_cell(gates, c):
    i, f, g, o = jnp.split(gates, 4, axis=-1)
    c = _sig(f) * c + _sig(i) * jnp.tanh(g)
    return _sig(o) * jnp.tanh(c), c


def _head(h, w0t, b0, w1t, b1):
    return _dot(jnp.maximum(_dot(h, w0t[...]) + b0[...], 0.0), w1t[...]) + b1[...]


def _lstm_head_body(h2_ref,
                    wih1t, whh1t, bb1, wih2t, whh2t, bb2,
                    sw0t, sb0, sw1t, sb1, dw0t, db0, dw1t, db1,
                    pw0t, pb0, pw1t, pb1,
                    s_out, d_out, p_out):
    x0 = h2_ref[:, :H]
    x1 = h2_ref[:, H:]
    g0 = _dot(x0, wih1t[...]) + bb1[...]
    y0, c = _lstm_cell(g0, 0.0)
    g1 = _dot(x1, wih1t[...]) + _dot(y0, whh1t[...]) + bb1[...]
    y1, c = _lstm_cell(g1, c)
    g0 = _dot(y0, wih2t[...]) + bb2[...]
    z0, c = _lstm_cell(g0, 0.0)
    g1 = _dot(y1, wih2t[...]) + _dot(z0, whh2t[...]) + bb2[...]
    z1, c = _lstm_cell(g1, c)
    s_out[...] = _head(z1, sw0t, sb0, sw1t, sb1)
    d = _head(z1, dw0t, db0, dw1t, db1)
    nrm = jnp.sqrt(jnp.sum(d * d, axis=-1, keepdims=True))
    d_out[...] = d / jnp.maximum(nrm, 1e-12)
    p_out[...] = _head(z1, pw0t, pb0, pw1t, pb1)


def _full_spec(shape):
    return pl.BlockSpec(shape, lambda i: tuple(0 for _ in shape))


def _row_spec(bn, cols):
    return pl.BlockSpec((bn, cols), lambda i: (i, 0))


def _tc_call(body, grid, in_specs, out_specs, out_shapes):
    return pl.pallas_call(
        body,
        grid=(grid,),
        in_specs=in_specs,
        out_specs=out_specs,
        out_shape=out_shapes,
    )


def _r2(v):
    return v.reshape(1, -1)


def _bd(wt):
    k, m = wt.shape
    z = jnp.zeros((k, m), wt.dtype)
    return jnp.concatenate(
        [jnp.concatenate([wt, z], axis=1), jnp.concatenate([z, wt], axis=1)], axis=0)


def _p2(v):
    return jnp.concatenate([v, v]).reshape(1, -1)


def kernel(x, edge_index, edge_attr, params):
    Bsz, T, N, INP = x.shape
    E = edge_index.shape[1]
    p = params
    pad = EP - E
    row2d = jnp.pad(edge_index[0], (0, pad)).reshape(EP // CH, CH)
    col2d = jnp.pad(edge_index[1], (0, pad)).reshape(EP // CH, CH)
    idx3 = jnp.stack([row2d, col2d])
    zeros_nh = jnp.zeros((NP, H2), F32)

    xsp = jnp.zeros((T, NP, INP), F32).at[:, :N].set(x[0])
    lyr0 = p['layers'][0]

    BN = 2048
    nblk = NP // BN
    BNL = 2000

    enc_in = [
        pl.BlockSpec((1, BN, INP), lambda i: (0, i, 0)),
        pl.BlockSpec((1, BN, INP), lambda i: (1, i, 0)),
        _full_spec((INP, H)), _full_spec((1, H)),
        _full_spec((H, H)), _full_spec((1, H)),
        _full_spec((H2, H2)), _full_spec((H2, H2)),
    ]
    enc_out = [_row_spec(BN, H2), pl.BlockSpec((2, BN, H2), lambda i: (0, i, 0))]
    h2, ab = _tc_call(
        _enc_body, nblk, enc_in, enc_out,
        [jax.ShapeDtypeStruct((NP, H2), F32),
         jax.ShapeDtypeStruct((2, NP, H2), F32)],
    )(xsp, xsp, p['encW0'].T, _r2(p['encb0']), p['encW1'].T, _r2(p['encb1']),
      _bd(lyr0['eW0'][:, :H].T), _bd(lyr0['eW0'][:, H:2 * H].T))

    EDIM = edge_attr.shape[1]
    ea8 = jnp.pad(edge_attr, ((0, pad), (0, 8 - EDIM)))

    BE = 8192
    eblk = EH // BE

    def edge_mlp(gab, lyr, half):
        w0et = jnp.pad(lyr['eW0'][:, 2 * H:].T, ((0, 8 - EDIM), (0, 0)))
        w0et2 = jnp.concatenate([w0et, w0et], axis=1)
        hb = half * (EH // BE)
        ga_spec = pl.BlockSpec((1, BE, H2), lambda i: (0, i, 0))
        gb_spec = pl.BlockSpec((1, BE, H2), lambda i: (1, i, 0))
        ea_spec = pl.BlockSpec((BE, 8), lambda i: (i + hb, 0))
        specs = [
            ga_spec, gb_spec, ea_spec,
            _full_spec((8, H2)), _full_spec((1, H2)),
            _full_spec((H2, H2)), _full_spec((1, H2)),
            _full_spec((H2, H2)), _full_spec((1, H2)),
        ]
        return _tc_call(
            functools.partial(_edge_body, be=BE, eoff=half * EH), eblk, specs,
            _row_spec(BE, H2),
            jax.ShapeDtypeStruct((EH, H2), F32),
        )(gab, gab, ea8, w0et2, _p2(lyr['eb0']), _bd(lyr['eW1'].T), _p2(lyr['eb1']),
          _bd(lyr['eW2'].T), _p2(lyr['eb2']))

    def node_update(h2c, parts0, parts1, lyr, nxt):
        w_common = (_bd(lyr['nW0'][:, :H].T), _bd(lyr['nW0'][:, H:].T), _p2(lyr['nb0']),
                    _bd(lyr['nW1'].T), _p2(lyr['nb1']), _r2(lyr['g']), _r2(lyr['be']))
        ag0_spec = pl.BlockSpec((1, BN, H2), lambda i: (0, i, 0))
        ag1_spec = pl.BlockSpec((1, BN, H2), lambda i: (1, i, 0))
        specs_common = [
            _row_spec(BN, H2), ag0_spec, ag1_spec, ag0_spec, ag1_spec,
            _full_spec((H2, H2)), _full_spec((H2, H2)), _full_spec((1, H2)),
            _full_spec((H2, H2)), _full_spec((1, H2)),
            _full_spec((1, H)), _full_spec((1, H)),
        ]
        if nxt is None:
            return _tc_call(
                _node_last_body, nblk, specs_common, _row_spec(BN, H2),
                jax.ShapeDtypeStruct((NP, H2), F32),
            )(h2c, parts0, parts0, parts1, parts1, *w_common)
        specs = specs_common + [_full_spec((H2, H2)), _full_spec((H2, H2))]
        out_specs = [_row_spec(BN, H2), pl.BlockSpec((2, BN, H2), lambda i: (0, i, 0))]
        return _tc_call(
            _node_proj_body, nblk, specs, out_specs,
            [jax.ShapeDtypeStruct((NP, H2), F32),
             jax.ShapeDtypeStruct((2, NP, H2), F32)],
        )(h2c, parts0, parts0, parts1, parts1, *w_common,
          _bd(nxt['eW0'][:, :H].T), _bd(nxt['eW0'][:, H:2 * H].T))

    EHC = EH // CH
    idxA = idx3[:, :EHC]
    idxB = idx3[:, EHC:]
    colA = col2d[:EHC].reshape(NW, NCHW_S, CH)
    colB = col2d[EHC:].reshape(NW, NCHW_S, CH)

    NL = len(p['layers'])
    for li, lyr in enumerate(p['layers']):
        nxt = p['layers'][li + 1] if li + 1 < NL else None
        gab0 = _sc_gather(ab, idxA)
        gab1 = _sc_gather(ab, idxB)
        e0 = edge_mlp(gab0, lyr, 0)
        e1 = edge_mlp(gab1, lyr, 1)
        parts0 = _sc_scatter(e0, colA, zeros_nh)
        parts1 = _sc_scatter(e1, colB, zeros_nh)
        if nxt is None:
            h2 = node_update(h2, parts0, parts1, lyr, None)
        else:
            h2, ab = node_update(h2, parts0, parts1, lyr, nxt)

    lp1, lp2 = p['lstm']
    specs = [
        _row_spec(BNL, H2),
        _full_spec((H, 4 * LAT)), _full_spec((LAT, 4 * LAT)), _full_spec((1, 4 * LAT)),
        _full_spec((LAT, 4 * LAT)), _full_spec((LAT, 4 * LAT)), _full_spec((1, 4 * LAT)),
        _full_spec((LAT, H // 2)), _full_spec((1, H // 2)), _full_spec((H // 2, 1)), _full_spec((1, 1)),
        _full_spec((LAT, H // 2)), _full_spec((1, H // 2)), _full_spec((H // 2, 2)), _full_spec((1, 2)),
        _full_spec((LAT, H // 2)), _full_spec((1, H // 2)), _full_spec((H // 2, 1)), _full_spec((1, 1)),
    ]
    out_specs = [_row_spec(BNL, 1), _row_spec(BNL, 2), _row_spec(BNL, 1)]
    out_shapes = [jax.ShapeDtypeStruct((N, 1), F32),
                  jax.ShapeDtypeStruct((N, 2), F32),
                  jax.ShapeDtypeStruct((N, 1), F32)]
    s, d, pp = _tc_call(_lstm_head_body, N // BNL, specs, out_specs, out_shapes)(
        h2,
        lp1['Wih'].T, lp1['Whh'].T, _r2(lp1['bih'] + lp1['bhh']),
        lp2['Wih'].T, lp2['Whh'].T, _r2(lp2['bih'] + lp2['bhh']),
        p['sW0'].T, _r2(p['sb0']), p['sW1'].T, _r2(p['sb1']),
        p['dW0'].T, _r2(p['db0']), p['dW1'].T, _r2(p['db1']),
        p['pW0'].T, _r2(p['pb0']), p['pW1'].T, _r2(p['pb1']),
    )
    return jnp.concatenate([s, d, pp], axis=-1).reshape(Bsz, N, 4)


# speedup vs baseline: 6.5109x; 1.0049x over previous
import functools

import jax
import jax.numpy as jnp
from jax import lax
from jax.experimental import pallas as pl
from jax.experimental.pallas import tpu as pltpu
from jax.experimental.pallas import tpu_sc as plsc

H = 64
H2 = 128
LAT = 128
F32 = jnp.float32

NC, NS = 2, 16
NW = NC * NS
CH = 128
CHG = 80

N_NODES = 10000
E_EDGES = 160000
EP = 163840
NCHW = EP // (NW * CH)
NCHT = EP // (NS * CH)
NP = 10240
NPT = NP // NS
EH = EP // 2
NCHT_G = EH // (NS * CHG)
NCHW_S = EH // (NW * CH)

DEPTH_G = 4
DEPTH_S = 2


def _dot(a, b):
    return jnp.dot(a, b, preferred_element_type=F32)


def _gather_body(tbl_hbm, idx_hbm, out_hbm, idxv, buf, spt, semi, semo):
    c = lax.axis_index("c")
    s = lax.axis_index("s")
    stg = pltpu.async_copy(tbl_hbm.at[c, pl.ds(s * NPT, NPT)],
                           spt.at[pl.ds(s * NPT, NPT)], semi.at[0])
    pltpu.sync_copy(idx_hbm.at[c, pl.ds(s * NCHT_G, NCHT_G)], idxv)
    stg.wait()
    plsc.subcore_barrier()

    for d in range(DEPTH_G - 1):
        pltpu.async_copy(spt.at[idxv.at[d]], buf.at[d], semi.at[d])

    @pl.loop(0, NCHT_G)
    def _(j):
        slot = lax.rem(j, DEPTH_G)
        pltpu.make_async_copy(spt.at[idxv.at[j]], buf.at[slot], semi.at[slot]).wait()
        base = (s * NCHT_G + j) * CHG
        pltpu.async_copy(buf.at[slot], out_hbm.at[c, pl.ds(base, CHG)], semo.at[slot])
        jn = j + DEPTH_G - 1

        @pl.when(jn < NCHT_G)
        def _():
            ns = lax.rem(jn, DEPTH_G)

            @pl.when(j > 0)
            def _():
                pbase = (s * NCHT_G + j - 1) * CHG
                pltpu.make_async_copy(
                    buf.at[ns], out_hbm.at[c, pl.ds(pbase, CHG)], semo.at[ns]).wait()

            pltpu.async_copy(spt.at[idxv.at[jn]], buf.at[ns], semi.at[ns])

    for k in range(NCHT_G - DEPTH_G, NCHT_G):
        slot = k % DEPTH_G
        base = (s * NCHT_G + k) * CHG
        pltpu.make_async_copy(buf.at[slot], out_hbm.at[c, pl.ds(base, CHG)],
                              semo.at[slot]).wait()


@functools.lru_cache(maxsize=None)
def _gather_call():
    return pl.kernel(
        _gather_body,
        out_type=jax.ShapeDtypeStruct((NC, EH, H2), F32),
        mesh=plsc.VectorSubcoreMesh(core_axis_name="c", subcore_axis_name="s"),
        scratch_types=[
            pltpu.VMEM((NCHT_G, CHG), jnp.int32),
            pltpu.VMEM((DEPTH_G, CHG, H2), F32),
            pltpu.VMEM_SHARED((NP, H2), F32),
            pltpu.SemaphoreType.DMA((DEPTH_G,)),
            pltpu.SemaphoreType.DMA((DEPTH_G,)),
        ],
    )


def _scatter_body(e2_hbm, col_hbm, zero_hbm, out_hbm, colv, bufe, aggr, sem):
    c = lax.axis_index("c")
    s = lax.axis_index("s")
    wid = s * NC + c
    cbase = wid * NCHW_S
    stg = pltpu.async_copy(zero_hbm.at[pl.ds(s * NPT, NPT)],
                           aggr.at[pl.ds(s * NPT, NPT)], sem.at[0])
    pltpu.sync_copy(col_hbm.at[wid], colv)
    stg.wait()
    plsc.subcore_barrier()

    for d in range(DEPTH_S - 1):
        pltpu.async_copy(e2_hbm.at[pl.ds((cbase + d) * CH, CH)], bufe.at[d], sem.at[d])

    @pl.loop(0, NCHW_S)
    def _(j):
        slot = lax.rem(j, DEPTH_S)
        pltpu.make_async_copy(
            e2_hbm.at[pl.ds((cbase + j) * CH, CH)], bufe.at[slot], sem.at[slot]
        ).wait()
        pltpu.sync_copy(bufe.at[slot], aggr.at[colv.at[j]], add=True)
        jn = j + DEPTH_S - 1

        @pl.when(jn < NCHW_S)
        def _():
            ns = lax.rem(jn, DEPTH_S)
            pltpu.async_copy(e2_hbm.at[pl.ds((cbase + jn) * CH, CH)], bufe.at[ns], sem.at[ns])

    plsc.subcore_barrier()
    pltpu.sync_copy(aggr.at[pl.ds(s * NPT, NPT)], out_hbm.at[c, pl.ds(s * NPT, NPT)])


@functools.lru_cache(maxsize=None)
def _scatter_call():
    return pl.kernel(
        _scatter_body,
        out_type=jax.ShapeDtypeStruct((NC, NP, H2), F32),
        mesh=plsc.VectorSubcoreMesh(core_axis_name="c", subcore_axis_name="s"),
        scratch_types=[
            pltpu.VMEM((NCHW_S, CH), jnp.int32),
            pltpu.VMEM((DEPTH_S, CH, H2), F32),
            pltpu.VMEM_SHARED((NP, H2), F32),
            pltpu.SemaphoreType.DMA((DEPTH_S,)),
        ],
    )


def _sc_gather(ab, idx3):
    return _gather_call()(ab, idx3)


def _sc_scatter(e2, col2d, zeros_nh):
    return _scatter_call()(e2, col2d, zeros_nh)


def _enc_body(x0_ref, x1_ref, w0t, b0, w1t, b1, wrt2, wct2, h_ref, ab_ref):
    def enc(xr):
        h = jnp.maximum(_dot(xr[0], w0t[...]) + b0[...], 0.0)
        return _dot(h, w1t[...]) + b1[...]

    h2 = jnp.concatenate([enc(x0_ref), enc(x1_ref)], axis=-1)
    h_ref[...] = h2
    ab_ref[0] = _dot(h2, wrt2[...])
    ab_ref[1] = _dot(h2, wct2[...])


def _edge_body(ga_ref, gb_ref, ea_ref, w0et2, b02, w1t2, b12, w2t2, b22,
               out_ref, *, be, eoff):
    e = ga_ref[0] + gb_ref[0] + _dot(ea_ref[...], w0et2[...]) + b02[...]
    e = jnp.maximum(e, 0.0)
    e = jnp.maximum(_dot(e, w1t2[...]) + b12[...], 0.0)
    e = _dot(e, w2t2[...]) + b22[...]
    gidx = lax.broadcasted_iota(jnp.int32, (be, 1), 0) + pl.program_id(0) * be + eoff
    out_ref[...] = jnp.where(gidx < E_EDGES, e, 0.0)


def _ln_half(hn, g, be):
    mu = jnp.mean(hn, axis=-1, keepdims=True)
    v = jnp.mean((hn - mu) ** 2, axis=-1, keepdims=True)
    return (hn - mu) / jnp.sqrt(v + 1e-5) * g + be


def _node_common(h_ref, ag0_ref, ag1_ref, ag2_ref, ag3_ref, w0ht2, w0at2, b02,
                 w1t2, b12, g, be):
    h2 = h_ref[...]
    ag = (ag0_ref[0] + ag1_ref[0]) + (ag2_ref[0] + ag3_ref[0])
    n = jnp.maximum(_dot(h2, w0ht2[...]) + _dot(ag, w0at2[...]) + b02[...], 0.0)
    n = _dot(n, w1t2[...]) + b12[...]
    hn = h2 + n
    return jnp.concatenate(
        [_ln_half(hn[:, :H], g[...], be[...]),
         _ln_half(hn[:, H:], g[...], be[...])], axis=-1)


def _node_proj_body(h_ref, ag0_ref, ag1_ref, ag2_ref, ag3_ref, w0ht2, w0at2, b02,
                    w1t2, b12, g, be, wrt2, wct2, h_out, ab_out):
    hnew = _node_common(h_ref, ag0_ref, ag1_ref, ag2_ref, ag3_ref, w0ht2, w0at2,
                        b02, w1t2, b12, g, be)
    h_out[...] = hnew
    ab_out[0] = _dot(hnew, wrt2[...])
    ab_out[1] = _dot(hnew, wct2[...])


def _node_last_body(h_ref, ag0_ref, ag1_ref, ag2_ref, ag3_ref, w0ht2, w0at2, b02,
                    w1t2, b12, g, be, h_out):
    h_out[...] = _node_common(h_ref, ag0_ref, ag1_ref, ag2_ref, ag3_ref, w0ht2,
                              w0at2, b02, w1t2, b12, g, be)


def _sig(x):
    return jax.nn.sigmoid(x)


def _lstm_cell(gates, c):
    i, f, g, o = jnp.split(gates, 4, axis=-1)
    c = _sig(f) * c + _sig(i) * jnp.tanh(g)
    return _sig(o) * jnp.tanh(c), c


def _head(h, w0t, b0, w1t, b1):
    return _dot(jnp.maximum(_dot(h, w0t[...]) + b0[...], 0.0), w1t[...]) + b1[...]


def _lstm_head_body(h2_ref,
                    wih1t, whh1t, bb1, wih2t, whh2t, bb2,
                    sw0t, sb0, sw1t, sb1, dw0t, db0, dw1t, db1,
                    pw0t, pb0, pw1t, pb1,
                    s_out, d_out, p_out):
    x0 = h2_ref[:, :H]
    x1 = h2_ref[:, H:]
    g0 = _dot(x0, wih1t[...]) + bb1[...]
    y0, c = _lstm_cell(g0, 0.0)
    g1 = _dot(x1, wih1t[...]) + _dot(y0, whh1t[...]) + bb1[...]
    y1, c = _lstm_cell(g1, c)
    g0 = _dot(y0, wih2t[...]) + bb2[...]
    z0, c = _lstm_cell(g0, 0.0)
    g1 = _dot(y1, wih2t[...]) + _dot(z0, whh2t[...]) + bb2[...]
    z1, c = _lstm_cell(g1, c)
    s_out[...] = _head(z1, sw0t, sb0, sw1t, sb1)
    d = _head(z1, dw0t, db0, dw1t, db1)
    nrm = jnp.sqrt(jnp.sum(d * d, axis=-1, keepdims=True))
    d_out[...] = d / jnp.maximum(nrm, 1e-12)
    p_out[...] = _head(z1, pw0t, pb0, pw1t, pb1)


def _full_spec(shape):
    return pl.BlockSpec(shape, lambda i: tuple(0 for _ in shape))


def _row_spec(bn, cols):
    return pl.BlockSpec((bn, cols), lambda i: (i, 0))


def _tc_call(body, grid, in_specs, out_specs, out_shapes):
    return pl.pallas_call(
        body,
        grid=(grid,),
        in_specs=in_specs,
        out_specs=out_specs,
        out_shape=out_shapes,
    )


def _r2(v):
    return v.reshape(1, -1)


def _bd(wt):
    k, m = wt.shape
    z = jnp.zeros((k, m), wt.dtype)
    return jnp.concatenate(
        [jnp.concatenate([wt, z], axis=1), jnp.concatenate([z, wt], axis=1)], axis=0)


def _p2(v):
    return jnp.concatenate([v, v]).reshape(1, -1)


def kernel(x, edge_index, edge_attr, params):
    Bsz, T, N, INP = x.shape
    E = edge_index.shape[1]
    p = params
    pad = EP - E
    row2d = jnp.pad(edge_index[0], (0, pad)).reshape(EP // CH, CH)
    col2d = jnp.pad(edge_index[1], (0, pad)).reshape(EP // CH, CH)
    idx3 = jnp.stack([row2d, col2d])
    zeros_nh = jnp.zeros((NP, H2), F32)

    xsp = jnp.zeros((T, NP, INP), F32).at[:, :N].set(x[0])
    lyr0 = p['layers'][0]

    BN = 2048
    nblk = NP // BN
    BNL = 2000

    enc_in = [
        pl.BlockSpec((1, BN, INP), lambda i: (0, i, 0)),
        pl.BlockSpec((1, BN, INP), lambda i: (1, i, 0)),
        _full_spec((INP, H)), _full_spec((1, H)),
        _full_spec((H, H)), _full_spec((1, H)),
        _full_spec((H2, H2)), _full_spec((H2, H2)),
    ]
    enc_out = [_row_spec(BN, H2), pl.BlockSpec((2, BN, H2), lambda i: (0, i, 0))]
    h2, ab = _tc_call(
        _enc_body, nblk, enc_in, enc_out,
        [jax.ShapeDtypeStruct((NP, H2), F32),
         jax.ShapeDtypeStruct((2, NP, H2), F32)],
    )(xsp, xsp, p['encW0'].T, _r2(p['encb0']), p['encW1'].T, _r2(p['encb1']),
      _bd(lyr0['eW0'][:, :H].T), _bd(lyr0['eW0'][:, H:2 * H].T))

    EDIM = edge_attr.shape[1]
    ea8 = jnp.pad(edge_attr, ((0, pad), (0, 8 - EDIM)))

    BE = 8192
    eblk = EH // BE

    def edge_mlp(gab, lyr, half):
        w0et = jnp.pad(lyr['eW0'][:, 2 * H:].T, ((0, 8 - EDIM), (0, 0)))
        w0et2 = jnp.concatenate([w0et, w0et], axis=1)
        hb = half * (EH // BE)
        ga_spec = pl.BlockSpec((1, BE, H2), lambda i: (0, i, 0))
        gb_spec = pl.BlockSpec((1, BE, H2), lambda i: (1, i, 0))
        ea_spec = pl.BlockSpec((BE, 8), lambda i: (i + hb, 0))
        specs = [
            ga_spec, gb_spec, ea_spec,
            _full_spec((8, H2)), _full_spec((1, H2)),
            _full_spec((H2, H2)), _full_spec((1, H2)),
            _full_spec((H2, H2)), _full_spec((1, H2)),
        ]
        return _tc_call(
            functools.partial(_edge_body, be=BE, eoff=half * EH), eblk, specs,
            _row_spec(BE, H2),
            jax.ShapeDtypeStruct((EH, H2), F32),
        )(gab, gab, ea8, w0et2, _p2(lyr['eb0']), _bd(lyr['eW1'].T), _p2(lyr['eb1']),
          _bd(lyr['eW2'].T), _p2(lyr['eb2']))

    def node_update(h2c, parts0, parts1, lyr, nxt):
        w_common = (_bd(lyr['nW0'][:, :H].T), _bd(lyr['nW0'][:, H:].T), _p2(lyr['nb0']),
                    _bd(lyr['nW1'].T), _p2(lyr['nb1']), _r2(lyr['g']), _r2(lyr['be']))
        ag0_spec = pl.BlockSpec((1, BN, H2), lambda i: (0, i, 0))
        ag1_spec = pl.BlockSpec((1, BN, H2), lambda i: (1, i, 0))
        specs_common = [
            _row_spec(BN, H2), ag0_spec, ag1_spec, ag0_spec, ag1_spec,
            _full_spec((H2, H2)), _full_spec((H2, H2)), _full_spec((1, H2)),
            _full_spec((H2, H2)), _full_spec((1, H2)),
            _full_spec((1, H)), _full_spec((1, H)),
        ]
        if nxt is None:
            return _tc_call(
                _node_last_body, nblk, specs_common, _row_spec(BN, H2),
                jax.ShapeDtypeStruct((NP, H2), F32),
            )(h2c, parts0, parts0, parts1, parts1, *w_common)
        specs = specs_common + [_full_spec((H2, H2)), _full_spec((H2, H2))]
        out_specs = [_row_spec(BN, H2), pl.BlockSpec((2, BN, H2), lambda i: (0, i, 0))]
        return _tc_call(
            _node_proj_body, nblk, specs, out_specs,
            [jax.ShapeDtypeStruct((NP, H2), F32),
             jax.ShapeDtypeStruct((2, NP, H2), F32)],
        )(h2c, parts0, parts0, parts1, parts1, *w_common,
          _bd(nxt['eW0'][:, :H].T), _bd(nxt['eW0'][:, H:2 * H].T))

    idxg = jnp.stack([jnp.pad(edge_index[0], (0, pad)),
                      jnp.pad(edge_index[1], (0, pad))]).reshape(2, EP // CHG, CHG)
    EHG = EH // CHG
    EHC = EH // CH
    idxA = idxg[:, :EHG]
    idxB = idxg[:, EHG:]
    colA = col2d[:EHC].reshape(NW, NCHW_S, CH)
    colB = col2d[EHC:].reshape(NW, NCHW_S, CH)

    NL = len(p['layers'])
    for li, lyr in enumerate(p['layers']):
        nxt = p['layers'][li + 1] if li + 1 < NL else None
        gab0 = _sc_gather(ab, idxA)
        gab1 = _sc_gather(ab, idxB)
        e0 = edge_mlp(gab0, lyr, 0)
        e1 = edge_mlp(gab1, lyr, 1)
        parts0 = _sc_scatter(e0, colA, zeros_nh)
        parts1 = _sc_scatter(e1, colB, zeros_nh)
        if nxt is None:
            h2 = node_update(h2, parts0, parts1, lyr, None)
        else:
            h2, ab = node_update(h2, parts0, parts1, lyr, nxt)

    lp1, lp2 = p['lstm']
    specs = [
        _row_spec(BNL, H2),
        _full_spec((H, 4 * LAT)), _full_spec((LAT, 4 * LAT)), _full_spec((1, 4 * LAT)),
        _full_spec((LAT, 4 * LAT)), _full_spec((LAT, 4 * LAT)), _full_spec((1, 4 * LAT)),
        _full_spec((LAT, H // 2)), _full_spec((1, H // 2)), _full_spec((H // 2, 1)), _full_spec((1, 1)),
        _full_spec((LAT, H // 2)), _full_spec((1, H // 2)), _full_spec((H // 2, 2)), _full_spec((1, 2)),
        _full_spec((LAT, H // 2)), _full_spec((1, H // 2)), _full_spec((H // 2, 1)), _full_spec((1, 1)),
    ]
    out_specs = [_row_spec(BNL, 1), _row_spec(BNL, 2), _row_spec(BNL, 1)]
    out_shapes = [jax.ShapeDtypeStruct((N, 1), F32),
                  jax.ShapeDtypeStruct((N, 2), F32),
                  jax.ShapeDtypeStruct((N, 1), F32)]
    s, d, pp = _tc_call(_lstm_head_body, N // BNL, specs, out_specs, out_shapes)(
        h2,
        lp1['Wih'].T, lp1['Whh'].T, _r2(lp1['bih'] + lp1['bhh']),
        lp2['Wih'].T, lp2['Whh'].T, _r2(lp2['bih'] + lp2['bhh']),
        p['sW0'].T, _r2(p['sb0']), p['sW1'].T, _r2(p['sb1']),
        p['dW0'].T, _r2(p['db0']), p['dW1'].T, _r2(p['db1']),
        p['pW0'].T, _r2(p['pb0']), p['pW1'].T, _r2(p['pb1']),
    )
    return jnp.concatenate([s, d, pp], axis=-1).reshape(Bsz, N, 4)
